# Initial kernel scaffold; baseline (speedup 1.0000x reference)
import jax
import jax.numpy as jnp
from jax.experimental import pallas as pl


def kernel(x, edge_index, homophily_cond, batch, params):
    raise NotImplementedError("write your pallas kernel here")


# speedup vs baseline: 1.1241x; 1.1241x over previous
import functools

import numpy as np
import jax
import jax.numpy as jnp
from jax import lax
from jax.experimental import pallas as pl
from jax.experimental.pallas import tpu as pltpu
from jax.experimental.pallas import tpu_sc as plsc

N = 10000
F = 256
LAT = 64
NCLS = 7

NC = 2
NS = 16
L = 16

_BN_S = float(1.0 / np.sqrt(1.0 + 1e-5))


NW = NC * NS
RPT = 320
AROWS = RPT + 8
NPAD = NW * RPT
ECHUNK = 128
MBUF = 288

def _sc_mesh():
    return plsc.VectorSubcoreMesh(core_axis_name="c", subcore_axis_name="s")


def _scan_chunk(dst_hbm, src_hbm, off, srcv, dstv, msrcv, mdstv, cnt, lo):
    pltpu.sync_copy(src_hbm.at[pl.ds(off, ECHUNK)], srcv)
    pltpu.sync_copy(dst_hbm.at[pl.ds(off, ECHUNK)], dstv)
    lane = lax.iota(jnp.int32, L)
    for j in range(ECHUNK // L):
        d16 = dstv[pl.ds(j * L, L)]
        ld = d16 - lo
        ok = (ld >= 0) & (ld < RPT)
        inc = plsc.cumsum(ok.astype(jnp.int32))
        pos = jnp.where(ok, cnt + inc - 1, MBUF - L + lane)
        plsc.store_scatter(msrcv, [pos], srcv[pl.ds(j * L, L)])
        plsc.store_scatter(mdstv, [pos], ld)
        cnt = cnt + plsc.all_reduce_population_count(ok)
    return cnt


def _pad_tail(msrcv, mdstv, cnt):
    for g in range(ECHUNK // L):
        lane = lax.iota(jnp.int32, L) + g * L
        m = lane < cnt
        msrcv[pl.ds(g * L, L)] = jnp.where(m, msrcv[pl.ds(g * L, L)], 0)
        mdstv[pl.ds(g * L, L)] = jnp.where(m, mdstv[pl.ds(g * L, L)], RPT)


def _shift_tail(msrcv, mdstv):
    for g in range(L - ECHUNK // L):
        sl = pl.ds(ECHUNK + g * L, L)
        dl = pl.ds(g * L, L)
        msrcv[dl] = msrcv[sl]
        mdstv[dl] = mdstv[sl]


def _deg_flush(mdstv, acc):
    one0 = jnp.where(lax.iota(jnp.int32, L) == 0, 1.0, 0.0)
    cols = lax.iota(jnp.int32, L)

    @pl.loop(0, ECHUNK)
    def _edges(e):
        row = plsc.load_gather(mdstv, [jnp.full((L,), e, jnp.int32)])
        plsc.addupdate_scatter(acc, [row, cols], one0)


def _deg_body(dst_hbm, src_hbm, zeros_hbm, out_hbm, srcv, dstv, msrcv, mdstv,
              acc):
    c = lax.axis_index("c")
    s = lax.axis_index("s")
    wid = c * NS + s
    lo = wid * RPT
    pltpu.sync_copy(zeros_hbm, acc)
    ep = dst_hbm.shape[0]

    @pl.loop(0, ep // ECHUNK, init_carry=jnp.zeros((L,), jnp.int32))
    def _chunks(i, cnt):
        cnt = _scan_chunk(dst_hbm, src_hbm, i * ECHUNK, srcv, dstv,
                          msrcv, mdstv, cnt, lo)
        full = jnp.any(cnt >= ECHUNK)

        @pl.when(full)
        def _flush():
            _deg_flush(mdstv, acc)
            _shift_tail(msrcv, mdstv)

        return jnp.where(full, cnt - ECHUNK, cnt)

    cnt = _chunks
    _pad_tail(msrcv, mdstv, cnt)
    _deg_flush(mdstv, acc)
    pltpu.sync_copy(acc.at[pl.ds(0, RPT)], out_hbm.at[pl.ds(lo, RPT)])


def _deg_call(src_pad, dst_pad, zeros_deg):
    return pl.kernel(
        _deg_body,
        out_type=jax.ShapeDtypeStruct((NPAD, L), jnp.float32),
        mesh=_sc_mesh(),
        compiler_params=pltpu.CompilerParams(needs_layout_passes=False),
        scratch_types=[
            pltpu.VMEM((ECHUNK,), jnp.int32),
            pltpu.VMEM((ECHUNK,), jnp.int32),
            pltpu.VMEM((MBUF,), jnp.int32),
            pltpu.VMEM((MBUF,), jnp.int32),
            pltpu.VMEM((AROWS, L), jnp.float32),
        ],
    )(dst_pad, src_pad, zeros_deg)


def _agg_flush(hw_hbm, msrcv, mdstv, gidx, rows, acc, sem):
    cols = lax.iota(jnp.int32, L)
    for g in range(ECHUNK // L):
        gidx[pl.ds(g * L, L)] = msrcv[pl.ds(g * L, L)]
    pltpu.async_copy(hw_hbm.at[gidx], rows, sem).wait()

    @pl.loop(0, ECHUNK)
    def _edges(e):
        row = plsc.load_gather(mdstv, [jnp.full((L,), e, jnp.int32)])
        for k in range(F // L):
            plsc.addupdate_scatter(acc, [row, cols + k * L],
                                   rows[e, pl.ds(k * L, L)])


def _agg_body(hw_hbm, src_hbm, dst_hbm, zeros_hbm, out_hbm,
              srcv, dstv, msrcv, mdstv, gidx, rows, acc, sem):
    c = lax.axis_index("c")
    s = lax.axis_index("s")
    wid = c * NS + s
    lo = wid * RPT
    pltpu.sync_copy(zeros_hbm, acc)
    ep = src_hbm.shape[0]

    @pl.loop(0, ep // ECHUNK, init_carry=jnp.zeros((L,), jnp.int32))
    def _chunks(i, cnt):
        cnt = _scan_chunk(dst_hbm, src_hbm, i * ECHUNK, srcv, dstv,
                          msrcv, mdstv, cnt, lo)
        full = jnp.any(cnt >= ECHUNK)

        @pl.when(full)
        def _flush():
            _agg_flush(hw_hbm, msrcv, mdstv, gidx, rows, acc, sem)
            _shift_tail(msrcv, mdstv)

        return jnp.where(full, cnt - ECHUNK, cnt)

    cnt = _chunks
    _pad_tail(msrcv, mdstv, cnt)
    _agg_flush(hw_hbm, msrcv, mdstv, gidx, rows, acc, sem)
    pltpu.sync_copy(acc.at[pl.ds(0, RPT)], out_hbm.at[pl.ds(lo, RPT)])


def _agg_call(hw, src_pad, dst_pad, zeros_acc):
    return pl.kernel(
        _agg_body,
        out_type=jax.ShapeDtypeStruct((NPAD, F), jnp.float32),
        mesh=_sc_mesh(),
        compiler_params=pltpu.CompilerParams(needs_layout_passes=False),
        scratch_types=[
            pltpu.VMEM((ECHUNK,), jnp.int32),
            pltpu.VMEM((ECHUNK,), jnp.int32),
            pltpu.VMEM((MBUF,), jnp.int32),
            pltpu.VMEM((MBUF,), jnp.int32),
            pltpu.VMEM((ECHUNK,), jnp.int32),
            pltpu.VMEM((ECHUNK, F), jnp.float32),
            pltpu.VMEM((AROWS, F), jnp.float32),
            pltpu.SemaphoreType.DMA,
        ],
    )(hw, src_pad, dst_pad, zeros_acc)


_RB = 1000


def _dis_of(degp_ref):
    deg = degp_ref[:, 0:1]
    return jnp.where(deg > 0.0, lax.rsqrt(deg), 0.0)


def _k2_body(x_ref, w_ref, degp_ref, out_ref):
    dis = _dis_of(degp_ref)
    out_ref[...] = jnp.dot(x_ref[...], w_ref[...],
                           preferred_element_type=jnp.float32) * dis


def _k2_call(x, w, degp):
    grid = (N // _RB,)
    return pl.pallas_call(
        _k2_body,
        grid=grid,
        in_specs=[
            pl.BlockSpec((_RB, F), lambda r: (r, 0)),
            pl.BlockSpec((F, F), lambda r: (0, 0)),
            pl.BlockSpec((_RB, L), lambda r: (r, 0)),
        ],
        out_specs=pl.BlockSpec((_RB, F), lambda r: (r, 0)),
        out_shape=jax.ShapeDtypeStruct((N, F), jnp.float32),
    )(x, w, degp)


def _post_h(hom_ref, agg_ref, degp_ref, b_ref, g_ref, bb_ref, hw_ref, hb_ref):
    dis = _dis_of(degp_ref)
    gcn = agg_ref[...] * dis + b_ref[...]
    h = jnp.maximum(gcn * _BN_S * g_ref[...] + bb_ref[...], 0.0)
    hv = (hom_ref[0] * hw_ref[0:1, :] + hom_ref[1] * hw_ref[1:2, :]
          + hom_ref[2] * hw_ref[2:3, :] + hb_ref[...])
    return h + hv, dis


def _k4_body(hom_ref, agg_ref, degp_ref, b_ref, g_ref, bb_ref, hw_ref, hb_ref,
             w2_ref, out_ref):
    h, dis = _post_h(hom_ref, agg_ref, degp_ref, b_ref, g_ref, bb_ref,
                     hw_ref, hb_ref)
    out_ref[...] = jnp.dot(h, w2_ref[...],
                           preferred_element_type=jnp.float32) * dis


def _k4_call(hom3, agg, degp, b, g, bb, hw, hb, w2):
    grid = (N // _RB,)
    row = lambda r: (r, 0)
    full2 = lambda r: (0, 0)
    return pl.pallas_call(
        _k4_body,
        grid=grid,
        in_specs=[
            pl.BlockSpec(memory_space=pltpu.SMEM),
            pl.BlockSpec((_RB, F), row),
            pl.BlockSpec((_RB, L), lambda r: (r, 0)),
            pl.BlockSpec((1, F), full2),
            pl.BlockSpec((1, F), full2),
            pl.BlockSpec((1, F), full2),
            pl.BlockSpec((3, F), full2),
            pl.BlockSpec((1, F), full2),
            pl.BlockSpec((F, F), full2),
        ],
        out_specs=pl.BlockSpec((_RB, F), row),
        out_shape=jax.ShapeDtypeStruct((N, F), jnp.float32),
    )(hom3, agg, degp, b, g, bb, hw, hb, w2)


def _k6_body(hom_ref, agg_ref, degp_ref, b_ref, g_ref, bb_ref, hw_ref, hb_ref,
             muw_ref, muw1_ref, mub_ref, lvw_ref, lvw1_ref, lvb_ref,
             mu_ref, lv_ref):
    h, _ = _post_h(hom_ref, agg_ref, degp_ref, b_ref, g_ref, bb_ref,
                   hw_ref, hb_ref)
    mucst = (hom_ref[0] * muw1_ref[0:1, :] + hom_ref[1] * muw1_ref[1:2, :]
             + hom_ref[2] * muw1_ref[2:3, :] + mub_ref[...])
    lvcst = (hom_ref[0] * lvw1_ref[0:1, :] + hom_ref[1] * lvw1_ref[1:2, :]
             + hom_ref[2] * lvw1_ref[2:3, :] + lvb_ref[...])
    mu_ref[...] = jnp.dot(h, muw_ref[...],
                          preferred_element_type=jnp.float32) + mucst
    lv_ref[...] = jnp.dot(h, lvw_ref[...],
                          preferred_element_type=jnp.float32) + lvcst


def _k6_call(hom3, agg, degp, b, g, bb, hw, hb, muw, muw1, mub, lvw, lvw1, lvb):
    grid = (N // _RB,)
    row = lambda r: (r, 0)
    full2 = lambda r: (0, 0)
    return pl.pallas_call(
        _k6_body,
        grid=grid,
        in_specs=[
            pl.BlockSpec(memory_space=pltpu.SMEM),
            pl.BlockSpec((_RB, F), row),
            pl.BlockSpec((_RB, L), lambda r: (r, 0)),
            pl.BlockSpec((1, F), full2),
            pl.BlockSpec((1, F), full2),
            pl.BlockSpec((1, F), full2),
            pl.BlockSpec((3, F), full2),
            pl.BlockSpec((1, F), full2),
            pl.BlockSpec((F, LAT), full2),
            pl.BlockSpec((3, LAT), full2),
            pl.BlockSpec((1, LAT), full2),
            pl.BlockSpec((F, LAT), full2),
            pl.BlockSpec((3, LAT), full2),
            pl.BlockSpec((1, LAT), full2),
        ],
        out_specs=[pl.BlockSpec((_RB, LAT), row), pl.BlockSpec((_RB, LAT), row)],
        out_shape=[jax.ShapeDtypeStruct((N, LAT), jnp.float32),
                   jax.ShapeDtypeStruct((N, LAT), jnp.float32)],
    )(hom3, agg, degp, b, g, bb, hw, hb, muw, muw1, mub, lvw, lvw1, lvb)


_ARB = 512
_ACB = 2048


def _adj_body(zi_ref, zj_ref, out_ref):
    prod = lax.dot_general(zi_ref[...], zj_ref[...], (((1,), (1,)), ((), ())),
                           preferred_element_type=jnp.float32)
    out_ref[...] = jax.nn.sigmoid(prod)


def _adj_call(z):
    grid = (pl.cdiv(N, _ARB), pl.cdiv(N, _ACB))
    return pl.pallas_call(
        _adj_body,
        grid=grid,
        in_specs=[
            pl.BlockSpec((_ARB, LAT), lambda r, c: (r, 0)),
            pl.BlockSpec((_ACB, LAT), lambda r, c: (c, 0)),
        ],
        out_specs=pl.BlockSpec((_ARB, _ACB), lambda r, c: (r, c)),
        out_shape=jax.ShapeDtypeStruct((N, N), jnp.float32),
    )(z, z)


def _k8_body(z_ref, pw_ref, pb_ref, pg_ref, pbb_ref, t1w_ref, t1b_ref,
             t2w_ref, t2b_ref, l1w_ref, l1b_ref, l2w_ref, l2b_ref,
             xr_ref, yl_ref):
    z = z_ref[...]
    zp = (jnp.dot(z, pw_ref[...], preferred_element_type=jnp.float32)
          + pb_ref[...]) * _BN_S * pg_ref[...] + pbb_ref[...]
    t = jnp.maximum(jnp.dot(zp, t1w_ref[...],
                            preferred_element_type=jnp.float32) + t1b_ref[...], 0.0)
    xr_ref[...] = jnp.dot(t, t2w_ref[...],
                          preferred_element_type=jnp.float32) + t2b_ref[...]
    u = jnp.maximum(jnp.dot(z, l1w_ref[...],
                            preferred_element_type=jnp.float32) + l1b_ref[...], 0.0)
    yl_ref[...] = jnp.dot(u, l2w_ref[...],
                          preferred_element_type=jnp.float32) + l2b_ref[...]


def _k8_call(z, pw, pb, pg, pbb, t1w, t1b, t2w, t2b, l1w, l1b, l2w, l2b):
    grid = (N // _RB,)
    row = lambda r: (r, 0)
    full2 = lambda r: (0, 0)
    tl = pw.shape[1]
    return pl.pallas_call(
        _k8_body,
        grid=grid,
        in_specs=[
            pl.BlockSpec((_RB, LAT), row),
            pl.BlockSpec((LAT, tl), full2),
            pl.BlockSpec((1, tl), full2),
            pl.BlockSpec((1, tl), full2),
            pl.BlockSpec((1, tl), full2),
            pl.BlockSpec((tl, F), full2),
            pl.BlockSpec((1, F), full2),
            pl.BlockSpec((F, F), full2),
            pl.BlockSpec((1, F), full2),
            pl.BlockSpec((LAT, LAT), full2),
            pl.BlockSpec((1, LAT), full2),
            pl.BlockSpec((LAT, 128), full2),
            pl.BlockSpec((1, 128), full2),
        ],
        out_specs=[pl.BlockSpec((_RB, F), row), pl.BlockSpec((_RB, 128), row)],
        out_shape=[jax.ShapeDtypeStruct((N, F), jnp.float32),
                   jax.ShapeDtypeStruct((N, 128), jnp.float32)],
    )(z, pw, pb, pg, pbb, t1w, t1b, t2w, t2b, l1w, l1b, l2w, l2b)


def _k9_body(z_ref, lh1_ref, lh1b_ref, lh2_ref, lh2b_ref,
             sh1_ref, sh1b_ref, sh2_ref, sh2b_ref,
             fh1_ref, fh1b_ref, fh2_ref, fh2b_ref,
             lh_ref, sh_ref, fh_ref):
    zg = jnp.sum(z_ref[...], axis=0, keepdims=True) * (1.0 / N)

    def head(w1, b1, w2, b2):
        u = jnp.maximum(jnp.dot(zg, w1[...],
                                preferred_element_type=jnp.float32) + b1[...], 0.0)
        return jnp.dot(u, w2[...], preferred_element_type=jnp.float32) + b2[...]

    lh_ref[...] = jax.nn.sigmoid(head(lh1_ref, lh1b_ref, lh2_ref, lh2b_ref))
    sh_ref[...] = jax.nn.sigmoid(head(sh1_ref, sh1b_ref, sh2_ref, sh2b_ref))
    fh_ref[...] = jnp.tanh(head(fh1_ref, fh1b_ref, fh2_ref, fh2b_ref))


def _k9_call(z, args):
    specs = [pl.BlockSpec((N, LAT), lambda: (0, 0))]
    for a in args:
        specs.append(pl.BlockSpec(a.shape, lambda: (0, 0)))
    return pl.pallas_call(
        _k9_body,
        in_specs=specs,
        out_specs=[pl.BlockSpec((1, 128), lambda: (0, 0))] * 3,
        out_shape=[jax.ShapeDtypeStruct((1, 128), jnp.float32)] * 3,
    )(z, *args)


def kernel(x, edge_index, homophily_cond, batch, params):
    p = params
    e = edge_index.shape[1]
    loop = jnp.arange(N, dtype=jnp.int32)
    src = jnp.concatenate([edge_index[0].astype(jnp.int32), loop])
    dst = jnp.concatenate([edge_index[1].astype(jnp.int32), loop])
    ep = ((e + N + ECHUNK - 1) // ECHUNK) * ECHUNK
    src = jnp.pad(src, (0, ep - e - N))
    dst = jnp.pad(dst, (0, ep - e - N), constant_values=N)

    zeros_deg = jnp.zeros((AROWS, L), jnp.float32)
    zeros_acc = jnp.zeros((AROWS, F), jnp.float32)
    hom3 = homophily_cond[0]

    r1 = lambda a: a.reshape(1, -1)

    degp = _deg_call(src, dst, zeros_deg)

    hw1 = _k2_call(x, p['gcn0_W'], degp)
    agg1 = _agg_call(hw1, src, dst, zeros_acc)[:N]

    hw2 = _k4_call(hom3, agg1, degp, r1(p['gcn0_b']), r1(p['bn0_g']),
                   r1(p['bn0_b']), p['hom0_W'], r1(p['hom0_b']), p['gcn1_W'])
    agg2 = _agg_call(hw2, src, dst, zeros_acc)[:N]

    muw0, muw1 = p['mu_W'][:F], p['mu_W'][F:]
    lvw0, lvw1 = p['lv_W'][:F], p['lv_W'][F:]
    mu, logvar = _k6_call(hom3, agg2, degp, r1(p['gcn1_b']), r1(p['bn1_g']),
                          r1(p['bn1_b']), p['hom1_W'], r1(p['hom1_b']),
                          muw0, muw1, r1(p['mu_b']), lvw0, lvw1, r1(p['lv_b']))
    z = mu

    adj = _adj_call(z)

    lab2w = jnp.pad(p['lab2_W'], ((0, 0), (0, 128 - NCLS)))
    lab2b = jnp.pad(r1(p['lab2_b']), ((0, 0), (0, 128 - NCLS)))
    x_recon, ylp = _k8_call(z, p['proj_W'], r1(p['proj_b']), r1(p['projbn_g']),
                            r1(p['projbn_b']), p['t1_W'], r1(p['t1_b']),
                            p['t2_W'], r1(p['t2_b']), p['lab1_W'],
                            r1(p['lab1_b']), lab2w, lab2b)
    y_logits = ylp[:, :NCLS]

    pad1 = lambda a: jnp.pad(r1(a), ((0, 0), (0, 128 - a.reshape(-1).shape[0])))
    head_args = (p['lh1_W'], r1(p['lh1_b']),
                 jnp.pad(p['lh2_W'], ((0, 0), (0, 127))), pad1(p['lh2_b']),
                 p['sh1_W'], r1(p['sh1_b']),
                 jnp.pad(p['sh2_W'], ((0, 0), (0, 127))), pad1(p['sh2_b']),
                 p['fh1_W'], r1(p['fh1_b']),
                 jnp.pad(p['fh2_W'], ((0, 0), (0, 127))), pad1(p['fh2_b']))
    lh, sh, fh = _k9_call(z, head_args)
    hom_pred = jnp.concatenate([lh[:, :1], sh[:, :1], fh[:, :1]], axis=1)

    return (adj, x_recon, y_logits, hom_pred, mu, logvar)


# speedup vs baseline: 2.8775x; 2.5598x over previous
import functools

import numpy as np
import jax
import jax.numpy as jnp
from jax import lax
from jax.experimental import pallas as pl
from jax.experimental.pallas import tpu as pltpu
from jax.experimental.pallas import tpu_sc as plsc

N = 10000
F = 256
LAT = 64
NCLS = 7

NC = 2
NS = 16
L = 16

_BN_S = float(1.0 / np.sqrt(1.0 + 1e-5))


NW = NC * NS
RPT = 320
AROWS = RPT + 8
NPAD = NW * RPT
ECHUNK = 128
SUP = 2688
MBUF = 288

def _sc_mesh():
    return plsc.VectorSubcoreMesh(core_axis_name="c", subcore_axis_name="s")


def _scan_chunk(srcv, dstv, off, msrcv, mdstv, cnt, lo):
    lane = lax.iota(jnp.int32, L)
    for j in range(ECHUNK // L):
        d16 = dstv[pl.ds(off + j * L, L)]
        ld = d16 - lo
        ok = (ld >= 0) & (ld < RPT)
        inc = plsc.cumsum(ok.astype(jnp.int32))
        pos = jnp.where(ok, cnt + inc - 1, MBUF - L + lane)
        plsc.store_scatter(msrcv, [pos], srcv[pl.ds(off + j * L, L)])
        plsc.store_scatter(mdstv, [pos], ld)
        cnt = cnt + plsc.all_reduce_population_count(ok)
    return cnt


def _pad_tail(msrcv, mdstv, cnt):
    for g in range(ECHUNK // L):
        lane = lax.iota(jnp.int32, L) + g * L
        m = lane < cnt
        msrcv[pl.ds(g * L, L)] = jnp.where(m, msrcv[pl.ds(g * L, L)], 0)
        mdstv[pl.ds(g * L, L)] = jnp.where(m, mdstv[pl.ds(g * L, L)], RPT)


def _shift_tail(msrcv, mdstv):
    for g in range(L - ECHUNK // L):
        sl = pl.ds(ECHUNK + g * L, L)
        dl = pl.ds(g * L, L)
        msrcv[dl] = msrcv[sl]
        mdstv[dl] = mdstv[sl]


def _deg_flush(mdstv, acc):
    one0 = jnp.where(lax.iota(jnp.int32, L) == 0, 1.0, 0.0)
    cols = lax.iota(jnp.int32, L)

    @pl.loop(0, ECHUNK)
    def _edges(e):
        row = plsc.load_gather(mdstv, [jnp.full((L,), e, jnp.int32)])
        plsc.addupdate_scatter(acc, [row, cols], one0)


def _deg_body(dst_hbm, src_hbm, zeros_hbm, out_hbm, srcv, dstv, msrcv, mdstv,
              acc):
    c = lax.axis_index("c")
    s = lax.axis_index("s")
    wid = c * NS + s
    lo = wid * RPT
    pltpu.sync_copy(zeros_hbm, acc)
    ep = dst_hbm.shape[0]

    @pl.loop(0, ep // SUP, init_carry=jnp.zeros((L,), jnp.int32))
    def _supers(i, cnt0):
        pltpu.sync_copy(src_hbm.at[pl.ds(i * SUP, SUP)], srcv)
        pltpu.sync_copy(dst_hbm.at[pl.ds(i * SUP, SUP)], dstv)

        @pl.loop(0, SUP // ECHUNK, init_carry=cnt0)
        def _chunks(k, cnt):
            cnt = _scan_chunk(srcv, dstv, k * ECHUNK, msrcv, mdstv, cnt, lo)
            full = jnp.any(cnt >= ECHUNK)

            @pl.when(full)
            def _flush():
                _deg_flush(mdstv, acc)
                _shift_tail(msrcv, mdstv)

            return jnp.where(full, cnt - ECHUNK, cnt)

        return _chunks

    cnt = _supers
    _pad_tail(msrcv, mdstv, cnt)
    _deg_flush(mdstv, acc)
    pltpu.sync_copy(acc.at[pl.ds(0, RPT)], out_hbm.at[pl.ds(lo, RPT)])


def _deg_call(src_pad, dst_pad, zeros_deg):
    return pl.kernel(
        _deg_body,
        out_type=jax.ShapeDtypeStruct((NPAD, L), jnp.float32),
        mesh=_sc_mesh(),
        compiler_params=pltpu.CompilerParams(needs_layout_passes=False),
        scratch_types=[
            pltpu.VMEM((SUP,), jnp.int32),
            pltpu.VMEM((SUP,), jnp.int32),
            pltpu.VMEM((MBUF,), jnp.int32),
            pltpu.VMEM((MBUF,), jnp.int32),
            pltpu.VMEM((AROWS, L), jnp.float32),
        ],
    )(dst_pad, src_pad, zeros_deg)


def _agg_flush(hw_hbm, msrcv, mdstv, gidx, rows, acc, sem):
    cols = lax.iota(jnp.int32, L)
    for g in range(ECHUNK // L):
        gidx[pl.ds(g * L, L)] = msrcv[pl.ds(g * L, L)]
    pltpu.async_copy(hw_hbm.at[gidx], rows, sem).wait()

    @pl.loop(0, ECHUNK)
    def _edges(e):
        row = plsc.load_gather(mdstv, [jnp.full((L,), e, jnp.int32)])
        for k in range(F // L):
            plsc.addupdate_scatter(acc, [row, cols + k * L],
                                   rows[e, pl.ds(k * L, L)])


def _agg_body(hw_hbm, src_hbm, dst_hbm, zeros_hbm, out_hbm,
              srcv, dstv, msrcv, mdstv, gidx, rows, acc, sem):
    c = lax.axis_index("c")
    s = lax.axis_index("s")
    wid = c * NS + s
    lo = wid * RPT
    pltpu.sync_copy(zeros_hbm, acc)
    ep = src_hbm.shape[0]

    @pl.loop(0, ep // SUP, init_carry=jnp.zeros((L,), jnp.int32))
    def _supers(i, cnt0):
        pltpu.sync_copy(src_hbm.at[pl.ds(i * SUP, SUP)], srcv)
        pltpu.sync_copy(dst_hbm.at[pl.ds(i * SUP, SUP)], dstv)

        @pl.loop(0, SUP // ECHUNK, init_carry=cnt0)
        def _chunks(k, cnt):
            cnt = _scan_chunk(srcv, dstv, k * ECHUNK, msrcv, mdstv, cnt, lo)
            full = jnp.any(cnt >= ECHUNK)

            @pl.when(full)
            def _flush():
                _agg_flush(hw_hbm, msrcv, mdstv, gidx, rows, acc, sem)
                _shift_tail(msrcv, mdstv)

            return jnp.where(full, cnt - ECHUNK, cnt)

        return _chunks

    cnt = _supers
    _pad_tail(msrcv, mdstv, cnt)
    _agg_flush(hw_hbm, msrcv, mdstv, gidx, rows, acc, sem)
    pltpu.sync_copy(acc.at[pl.ds(0, RPT)], out_hbm.at[pl.ds(lo, RPT)])


def _agg_call(hw, src_pad, dst_pad, zeros_acc):
    return pl.kernel(
        _agg_body,
        out_type=jax.ShapeDtypeStruct((NPAD, F), jnp.float32),
        mesh=_sc_mesh(),
        compiler_params=pltpu.CompilerParams(needs_layout_passes=False),
        scratch_types=[
            pltpu.VMEM((SUP,), jnp.int32),
            pltpu.VMEM((SUP,), jnp.int32),
            pltpu.VMEM((MBUF,), jnp.int32),
            pltpu.VMEM((MBUF,), jnp.int32),
            pltpu.VMEM((ECHUNK,), jnp.int32),
            pltpu.VMEM((ECHUNK, F), jnp.float32),
            pltpu.VMEM((AROWS, F), jnp.float32),
            pltpu.SemaphoreType.DMA,
        ],
    )(hw, src_pad, dst_pad, zeros_acc)


_RB = 1000


def _dis_of(degp_ref):
    deg = degp_ref[:, 0:1]
    return jnp.where(deg > 0.0, lax.rsqrt(deg), 0.0)


def _k2_body(x_ref, w_ref, degp_ref, out_ref):
    dis = _dis_of(degp_ref)
    out_ref[...] = jnp.dot(x_ref[...], w_ref[...],
                           preferred_element_type=jnp.float32) * dis


def _k2_call(x, w, degp):
    grid = (N // _RB,)
    return pl.pallas_call(
        _k2_body,
        grid=grid,
        in_specs=[
            pl.BlockSpec((_RB, F), lambda r: (r, 0)),
            pl.BlockSpec((F, F), lambda r: (0, 0)),
            pl.BlockSpec((_RB, L), lambda r: (r, 0)),
        ],
        out_specs=pl.BlockSpec((_RB, F), lambda r: (r, 0)),
        out_shape=jax.ShapeDtypeStruct((N, F), jnp.float32),
    )(x, w, degp)


def _post_h(hom_ref, agg_ref, degp_ref, b_ref, g_ref, bb_ref, hw_ref, hb_ref):
    dis = _dis_of(degp_ref)
    gcn = agg_ref[...] * dis + b_ref[...]
    h = jnp.maximum(gcn * _BN_S * g_ref[...] + bb_ref[...], 0.0)
    hv = (hom_ref[0] * hw_ref[0:1, :] + hom_ref[1] * hw_ref[1:2, :]
          + hom_ref[2] * hw_ref[2:3, :] + hb_ref[...])
    return h + hv, dis


def _k4_body(hom_ref, agg_ref, degp_ref, b_ref, g_ref, bb_ref, hw_ref, hb_ref,
             w2_ref, out_ref):
    h, dis = _post_h(hom_ref, agg_ref, degp_ref, b_ref, g_ref, bb_ref,
                     hw_ref, hb_ref)
    out_ref[...] = jnp.dot(h, w2_ref[...],
                           preferred_element_type=jnp.float32) * dis


def _k4_call(hom3, agg, degp, b, g, bb, hw, hb, w2):
    grid = (N // _RB,)
    row = lambda r: (r, 0)
    full2 = lambda r: (0, 0)
    return pl.pallas_call(
        _k4_body,
        grid=grid,
        in_specs=[
            pl.BlockSpec(memory_space=pltpu.SMEM),
            pl.BlockSpec((_RB, F), row),
            pl.BlockSpec((_RB, L), lambda r: (r, 0)),
            pl.BlockSpec((1, F), full2),
            pl.BlockSpec((1, F), full2),
            pl.BlockSpec((1, F), full2),
            pl.BlockSpec((3, F), full2),
            pl.BlockSpec((1, F), full2),
            pl.BlockSpec((F, F), full2),
        ],
        out_specs=pl.BlockSpec((_RB, F), row),
        out_shape=jax.ShapeDtypeStruct((N, F), jnp.float32),
    )(hom3, agg, degp, b, g, bb, hw, hb, w2)


def _k6_body(hom_ref, agg_ref, degp_ref, b_ref, g_ref, bb_ref, hw_ref, hb_ref,
             muw_ref, muw1_ref, mub_ref, lvw_ref, lvw1_ref, lvb_ref,
             mu_ref, lv_ref):
    h, _ = _post_h(hom_ref, agg_ref, degp_ref, b_ref, g_ref, bb_ref,
                   hw_ref, hb_ref)
    mucst = (hom_ref[0] * muw1_ref[0:1, :] + hom_ref[1] * muw1_ref[1:2, :]
             + hom_ref[2] * muw1_ref[2:3, :] + mub_ref[...])
    lvcst = (hom_ref[0] * lvw1_ref[0:1, :] + hom_ref[1] * lvw1_ref[1:2, :]
             + hom_ref[2] * lvw1_ref[2:3, :] + lvb_ref[...])
    mu_ref[...] = jnp.dot(h, muw_ref[...],
                          preferred_element_type=jnp.float32) + mucst
    lv_ref[...] = jnp.dot(h, lvw_ref[...],
                          preferred_element_type=jnp.float32) + lvcst


def _k6_call(hom3, agg, degp, b, g, bb, hw, hb, muw, muw1, mub, lvw, lvw1, lvb):
    grid = (N // _RB,)
    row = lambda r: (r, 0)
    full2 = lambda r: (0, 0)
    return pl.pallas_call(
        _k6_body,
        grid=grid,
        in_specs=[
            pl.BlockSpec(memory_space=pltpu.SMEM),
            pl.BlockSpec((_RB, F), row),
            pl.BlockSpec((_RB, L), lambda r: (r, 0)),
            pl.BlockSpec((1, F), full2),
            pl.BlockSpec((1, F), full2),
            pl.BlockSpec((1, F), full2),
            pl.BlockSpec((3, F), full2),
            pl.BlockSpec((1, F), full2),
            pl.BlockSpec((F, LAT), full2),
            pl.BlockSpec((3, LAT), full2),
            pl.BlockSpec((1, LAT), full2),
            pl.BlockSpec((F, LAT), full2),
            pl.BlockSpec((3, LAT), full2),
            pl.BlockSpec((1, LAT), full2),
        ],
        out_specs=[pl.BlockSpec((_RB, LAT), row), pl.BlockSpec((_RB, LAT), row)],
        out_shape=[jax.ShapeDtypeStruct((N, LAT), jnp.float32),
                   jax.ShapeDtypeStruct((N, LAT), jnp.float32)],
    )(hom3, agg, degp, b, g, bb, hw, hb, muw, muw1, mub, lvw, lvw1, lvb)


_ARB = 512
_ACB = 2048


def _adj_body(zi_ref, zj_ref, out_ref):
    prod = lax.dot_general(zi_ref[...], zj_ref[...], (((1,), (1,)), ((), ())),
                           preferred_element_type=jnp.float32)
    out_ref[...] = jax.nn.sigmoid(prod)


def _adj_call(z):
    grid = (pl.cdiv(N, _ARB), pl.cdiv(N, _ACB))
    return pl.pallas_call(
        _adj_body,
        grid=grid,
        in_specs=[
            pl.BlockSpec((_ARB, LAT), lambda r, c: (r, 0)),
            pl.BlockSpec((_ACB, LAT), lambda r, c: (c, 0)),
        ],
        out_specs=pl.BlockSpec((_ARB, _ACB), lambda r, c: (r, c)),
        out_shape=jax.ShapeDtypeStruct((N, N), jnp.float32),
    )(z, z)


def _k8_body(z_ref, pw_ref, pb_ref, pg_ref, pbb_ref, t1w_ref, t1b_ref,
             t2w_ref, t2b_ref, l1w_ref, l1b_ref, l2w_ref, l2b_ref,
             xr_ref, yl_ref):
    z = z_ref[...]
    zp = (jnp.dot(z, pw_ref[...], preferred_element_type=jnp.float32)
          + pb_ref[...]) * _BN_S * pg_ref[...] + pbb_ref[...]
    t = jnp.maximum(jnp.dot(zp, t1w_ref[...],
                            preferred_element_type=jnp.float32) + t1b_ref[...], 0.0)
    xr_ref[...] = jnp.dot(t, t2w_ref[...],
                          preferred_element_type=jnp.float32) + t2b_ref[...]
    u = jnp.maximum(jnp.dot(z, l1w_ref[...],
                            preferred_element_type=jnp.float32) + l1b_ref[...], 0.0)
    yl_ref[...] = jnp.dot(u, l2w_ref[...],
                          preferred_element_type=jnp.float32) + l2b_ref[...]


def _k8_call(z, pw, pb, pg, pbb, t1w, t1b, t2w, t2b, l1w, l1b, l2w, l2b):
    grid = (N // _RB,)
    row = lambda r: (r, 0)
    full2 = lambda r: (0, 0)
    tl = pw.shape[1]
    return pl.pallas_call(
        _k8_body,
        grid=grid,
        in_specs=[
            pl.BlockSpec((_RB, LAT), row),
            pl.BlockSpec((LAT, tl), full2),
            pl.BlockSpec((1, tl), full2),
            pl.BlockSpec((1, tl), full2),
            pl.BlockSpec((1, tl), full2),
            pl.BlockSpec((tl, F), full2),
            pl.BlockSpec((1, F), full2),
            pl.BlockSpec((F, F), full2),
            pl.BlockSpec((1, F), full2),
            pl.BlockSpec((LAT, LAT), full2),
            pl.BlockSpec((1, LAT), full2),
            pl.BlockSpec((LAT, 128), full2),
            pl.BlockSpec((1, 128), full2),
        ],
        out_specs=[pl.BlockSpec((_RB, F), row), pl.BlockSpec((_RB, 128), row)],
        out_shape=[jax.ShapeDtypeStruct((N, F), jnp.float32),
                   jax.ShapeDtypeStruct((N, 128), jnp.float32)],
    )(z, pw, pb, pg, pbb, t1w, t1b, t2w, t2b, l1w, l1b, l2w, l2b)


def _k9_body(z_ref, lh1_ref, lh1b_ref, lh2_ref, lh2b_ref,
             sh1_ref, sh1b_ref, sh2_ref, sh2b_ref,
             fh1_ref, fh1b_ref, fh2_ref, fh2b_ref,
             lh_ref, sh_ref, fh_ref):
    zg = jnp.sum(z_ref[...], axis=0, keepdims=True) * (1.0 / N)

    def head(w1, b1, w2, b2):
        u = jnp.maximum(jnp.dot(zg, w1[...],
                                preferred_element_type=jnp.float32) + b1[...], 0.0)
        return jnp.dot(u, w2[...], preferred_element_type=jnp.float32) + b2[...]

    lh_ref[...] = jax.nn.sigmoid(head(lh1_ref, lh1b_ref, lh2_ref, lh2b_ref))
    sh_ref[...] = jax.nn.sigmoid(head(sh1_ref, sh1b_ref, sh2_ref, sh2b_ref))
    fh_ref[...] = jnp.tanh(head(fh1_ref, fh1b_ref, fh2_ref, fh2b_ref))


def _k9_call(z, args):
    specs = [pl.BlockSpec((N, LAT), lambda: (0, 0))]
    for a in args:
        specs.append(pl.BlockSpec(a.shape, lambda: (0, 0)))
    return pl.pallas_call(
        _k9_body,
        in_specs=specs,
        out_specs=[pl.BlockSpec((1, 128), lambda: (0, 0))] * 3,
        out_shape=[jax.ShapeDtypeStruct((1, 128), jnp.float32)] * 3,
    )(z, *args)


def kernel(x, edge_index, homophily_cond, batch, params):
    p = params
    e = edge_index.shape[1]
    loop = jnp.arange(N, dtype=jnp.int32)
    src = jnp.concatenate([edge_index[0].astype(jnp.int32), loop])
    dst = jnp.concatenate([edge_index[1].astype(jnp.int32), loop])
    ep = ((e + N + SUP - 1) // SUP) * SUP
    src = jnp.pad(src, (0, ep - e - N))
    dst = jnp.pad(dst, (0, ep - e - N), constant_values=N)

    zeros_deg = jnp.zeros((AROWS, L), jnp.float32)
    zeros_acc = jnp.zeros((AROWS, F), jnp.float32)
    hom3 = homophily_cond[0]

    r1 = lambda a: a.reshape(1, -1)

    degp = _deg_call(src, dst, zeros_deg)

    hw1 = _k2_call(x, p['gcn0_W'], degp)
    agg1 = _agg_call(hw1, src, dst, zeros_acc)[:N]

    hw2 = _k4_call(hom3, agg1, degp, r1(p['gcn0_b']), r1(p['bn0_g']),
                   r1(p['bn0_b']), p['hom0_W'], r1(p['hom0_b']), p['gcn1_W'])
    agg2 = _agg_call(hw2, src, dst, zeros_acc)[:N]

    muw0, muw1 = p['mu_W'][:F], p['mu_W'][F:]
    lvw0, lvw1 = p['lv_W'][:F], p['lv_W'][F:]
    mu, logvar = _k6_call(hom3, agg2, degp, r1(p['gcn1_b']), r1(p['bn1_g']),
                          r1(p['bn1_b']), p['hom1_W'], r1(p['hom1_b']),
                          muw0, muw1, r1(p['mu_b']), lvw0, lvw1, r1(p['lv_b']))
    z = mu

    adj = _adj_call(z)

    lab2w = jnp.pad(p['lab2_W'], ((0, 0), (0, 128 - NCLS)))
    lab2b = jnp.pad(r1(p['lab2_b']), ((0, 0), (0, 128 - NCLS)))
    x_recon, ylp = _k8_call(z, p['proj_W'], r1(p['proj_b']), r1(p['projbn_g']),
                            r1(p['projbn_b']), p['t1_W'], r1(p['t1_b']),
                            p['t2_W'], r1(p['t2_b']), p['lab1_W'],
                            r1(p['lab1_b']), lab2w, lab2b)
    y_logits = ylp[:, :NCLS]

    pad1 = lambda a: jnp.pad(r1(a), ((0, 0), (0, 128 - a.reshape(-1).shape[0])))
    head_args = (p['lh1_W'], r1(p['lh1_b']),
                 jnp.pad(p['lh2_W'], ((0, 0), (0, 127))), pad1(p['lh2_b']),
                 p['sh1_W'], r1(p['sh1_b']),
                 jnp.pad(p['sh2_W'], ((0, 0), (0, 127))), pad1(p['sh2_b']),
                 p['fh1_W'], r1(p['fh1_b']),
                 jnp.pad(p['fh2_W'], ((0, 0), (0, 127))), pad1(p['fh2_b']))
    lh, sh, fh = _k9_call(z, head_args)
    hom_pred = jnp.concatenate([lh[:, :1], sh[:, :1], fh[:, :1]], axis=1)

    return (adj, x_recon, y_logits, hom_pred, mu, logvar)


# speedup vs baseline: 3.4710x; 1.2063x over previous
import functools

import numpy as np
import jax
import jax.numpy as jnp
from jax import lax
from jax.experimental import pallas as pl
from jax.experimental.pallas import tpu as pltpu
from jax.experimental.pallas import tpu_sc as plsc

N = 10000
F = 256
LAT = 64
NCLS = 7

NC = 2
NS = 16
L = 16

_BN_S = float(1.0 / np.sqrt(1.0 + 1e-5))


NW = NC * NS
RPT = 320
AROWS = RPT + 8
NPAD = NW * RPT
ECHUNK = 128
SUP = 2688
MBUF = 288
def _cap(ep):
    return ep + ECHUNK

def _sc_mesh():
    return plsc.VectorSubcoreMesh(core_axis_name="c", subcore_axis_name="s")


def _scan_chunk(srcv, dstv, off, msrcv, mdstv, cnt, lo):
    lane = lax.iota(jnp.int32, L)
    for j in range(ECHUNK // L):
        d16 = dstv[pl.ds(off + j * L, L)]
        ld = d16 - lo
        ok = (ld >= 0) & (ld < RPT)
        inc = plsc.cumsum(ok.astype(jnp.int32))
        pos = jnp.where(ok, cnt + inc - 1, MBUF - L + lane)
        plsc.store_scatter(msrcv, [pos], srcv[pl.ds(off + j * L, L)])
        plsc.store_scatter(mdstv, [pos], ld)
        cnt = cnt + plsc.all_reduce_population_count(ok)
    return cnt


def _pad_tail(msrcv, mdstv, cnt):
    for g in range(ECHUNK // L):
        lane = lax.iota(jnp.int32, L) + g * L
        m = lane < cnt
        msrcv[pl.ds(g * L, L)] = jnp.where(m, msrcv[pl.ds(g * L, L)], 0)
        mdstv[pl.ds(g * L, L)] = jnp.where(m, mdstv[pl.ds(g * L, L)], RPT)


def _shift_tail(msrcv, mdstv):
    for g in range(L - ECHUNK // L):
        sl = pl.ds(ECHUNK + g * L, L)
        dl = pl.ds(g * L, L)
        msrcv[dl] = msrcv[sl]
        mdstv[dl] = mdstv[sl]


def _deg_flush(mdstv, acc):
    one0 = jnp.where(lax.iota(jnp.int32, L) == 0, 1.0, 0.0)
    cols = lax.iota(jnp.int32, L)

    @pl.loop(0, ECHUNK)
    def _edges(e):
        row = plsc.load_gather(mdstv, [jnp.full((L,), e, jnp.int32)])
        plsc.addupdate_scatter(acc, [row, cols], one0)


def _deg_body(dst_hbm, src_hbm, zeros_hbm, out_hbm, srcl_hbm, dstl_hbm,
              cntl_hbm, srcv, dstv, msrcv, mdstv, cntbuf, acc):
    c = lax.axis_index("c")
    s = lax.axis_index("s")
    wid = c * NS + s
    lo = wid * RPT
    pltpu.sync_copy(zeros_hbm, acc)
    ep = dst_hbm.shape[0]

    def _emit(nf):
        _deg_flush(mdstv, acc)
        pltpu.sync_copy(msrcv.at[pl.ds(0, ECHUNK)],
                        srcl_hbm.at[wid, pl.ds(nf * ECHUNK, ECHUNK)])
        pltpu.sync_copy(mdstv.at[pl.ds(0, ECHUNK)],
                        dstl_hbm.at[wid, pl.ds(nf * ECHUNK, ECHUNK)])

    @pl.loop(0, ep // SUP, init_carry=(jnp.zeros((L,), jnp.int32), 0))
    def _supers(i, carry):
        cnt0, nf0 = carry
        pltpu.sync_copy(src_hbm.at[pl.ds(i * SUP, SUP)], srcv)
        pltpu.sync_copy(dst_hbm.at[pl.ds(i * SUP, SUP)], dstv)

        @pl.loop(0, SUP // ECHUNK, init_carry=(cnt0, nf0))
        def _chunks(k, carry2):
            cnt, nf = carry2
            cnt = _scan_chunk(srcv, dstv, k * ECHUNK, msrcv, mdstv, cnt, lo)
            full = jnp.any(cnt >= ECHUNK)

            @pl.when(full)
            def _flush():
                _emit(nf)
                _shift_tail(msrcv, mdstv)

            return (jnp.where(full, cnt - ECHUNK, cnt),
                    jnp.where(full, nf + 1, nf))

        return _chunks

    cnt, nf = _supers
    _pad_tail(msrcv, mdstv, cnt)
    _emit(nf)
    cntbuf[...] = jnp.full((L,), (nf + 1) * ECHUNK, jnp.int32)
    pltpu.sync_copy(cntbuf, cntl_hbm.at[wid])
    pltpu.sync_copy(acc.at[pl.ds(0, RPT)], out_hbm.at[pl.ds(lo, RPT)])


def _deg_call(src_pad, dst_pad, zeros_deg):
    ep = src_pad.shape[0]
    cap = _cap(ep)
    return pl.kernel(
        _deg_body,
        out_type=[jax.ShapeDtypeStruct((NPAD, L), jnp.float32),
                  jax.ShapeDtypeStruct((NW, cap), jnp.int32),
                  jax.ShapeDtypeStruct((NW, cap), jnp.int32),
                  jax.ShapeDtypeStruct((NW, L), jnp.int32)],
        mesh=_sc_mesh(),
        compiler_params=pltpu.CompilerParams(needs_layout_passes=False),
        scratch_types=[
            pltpu.VMEM((SUP,), jnp.int32),
            pltpu.VMEM((SUP,), jnp.int32),
            pltpu.VMEM((MBUF,), jnp.int32),
            pltpu.VMEM((MBUF,), jnp.int32),
            pltpu.VMEM((L,), jnp.int32),
            pltpu.VMEM((AROWS, L), jnp.float32),
        ],
    )(dst_pad, src_pad, zeros_deg)


def _agg_flush(hw_hbm, mdstv, gidx, rows, acc, sem):
    cols = lax.iota(jnp.int32, L)
    pltpu.async_copy(hw_hbm.at[gidx], rows, sem).wait()

    @pl.loop(0, ECHUNK)
    def _edges(e):
        row = plsc.load_gather(mdstv, [jnp.full((L,), e, jnp.int32)])
        for k in range(F // L):
            plsc.addupdate_scatter(acc, [row, cols + k * L],
                                   rows[e, pl.ds(k * L, L)])


def _agg_body(hw_hbm, srcl_hbm, dstl_hbm, cntl_hbm, zeros_hbm, out_hbm,
              cntv, mdstv, gidx, rows, acc, sem):
    c = lax.axis_index("c")
    s = lax.axis_index("s")
    wid = c * NS + s
    lo = wid * RPT
    pltpu.sync_copy(zeros_hbm, acc)
    pltpu.sync_copy(cntl_hbm.at[wid], cntv)
    cnt = cntv[...]
    cap = srcl_hbm.shape[1]

    @pl.loop(0, cap // ECHUNK)
    def _blocks(i):
        @pl.when(jnp.any(i * ECHUNK < cnt))
        def _do():
            pltpu.sync_copy(srcl_hbm.at[wid, pl.ds(i * ECHUNK, ECHUNK)], gidx)
            pltpu.sync_copy(dstl_hbm.at[wid, pl.ds(i * ECHUNK, ECHUNK)], mdstv)
            _agg_flush(hw_hbm, mdstv, gidx, rows, acc, sem)

    pltpu.sync_copy(acc.at[pl.ds(0, RPT)], out_hbm.at[pl.ds(lo, RPT)])


def _agg_call(hw, srcl, dstl, cntl, zeros_acc):
    return pl.kernel(
        _agg_body,
        out_type=jax.ShapeDtypeStruct((NPAD, F), jnp.float32),
        mesh=_sc_mesh(),
        compiler_params=pltpu.CompilerParams(needs_layout_passes=False),
        scratch_types=[
            pltpu.VMEM((L,), jnp.int32),
            pltpu.VMEM((ECHUNK,), jnp.int32),
            pltpu.VMEM((ECHUNK,), jnp.int32),
            pltpu.VMEM((ECHUNK, F), jnp.float32),
            pltpu.VMEM((AROWS, F), jnp.float32),
            pltpu.SemaphoreType.DMA,
        ],
    )(hw, srcl, dstl, cntl, zeros_acc)


_RB = 1000


def _dis_of(degp_ref):
    deg = degp_ref[:, 0:1]
    return jnp.where(deg > 0.0, lax.rsqrt(deg), 0.0)


def _k2_body(x_ref, w_ref, degp_ref, out_ref):
    dis = _dis_of(degp_ref)
    out_ref[...] = jnp.dot(x_ref[...], w_ref[...],
                           preferred_element_type=jnp.float32) * dis


def _k2_call(x, w, degp):
    grid = (N // _RB,)
    return pl.pallas_call(
        _k2_body,
        grid=grid,
        in_specs=[
            pl.BlockSpec((_RB, F), lambda r: (r, 0)),
            pl.BlockSpec((F, F), lambda r: (0, 0)),
            pl.BlockSpec((_RB, L), lambda r: (r, 0)),
        ],
        out_specs=pl.BlockSpec((_RB, F), lambda r: (r, 0)),
        out_shape=jax.ShapeDtypeStruct((N, F), jnp.float32),
    )(x, w, degp)


def _post_h(hom_ref, agg_ref, degp_ref, b_ref, g_ref, bb_ref, hw_ref, hb_ref):
    dis = _dis_of(degp_ref)
    gcn = agg_ref[...] * dis + b_ref[...]
    h = jnp.maximum(gcn * _BN_S * g_ref[...] + bb_ref[...], 0.0)
    hv = (hom_ref[0] * hw_ref[0:1, :] + hom_ref[1] * hw_ref[1:2, :]
          + hom_ref[2] * hw_ref[2:3, :] + hb_ref[...])
    return h + hv, dis


def _k4_body(hom_ref, agg_ref, degp_ref, b_ref, g_ref, bb_ref, hw_ref, hb_ref,
             w2_ref, out_ref):
    h, dis = _post_h(hom_ref, agg_ref, degp_ref, b_ref, g_ref, bb_ref,
                     hw_ref, hb_ref)
    out_ref[...] = jnp.dot(h, w2_ref[...],
                           preferred_element_type=jnp.float32) * dis


def _k4_call(hom3, agg, degp, b, g, bb, hw, hb, w2):
    grid = (N // _RB,)
    row = lambda r: (r, 0)
    full2 = lambda r: (0, 0)
    return pl.pallas_call(
        _k4_body,
        grid=grid,
        in_specs=[
            pl.BlockSpec(memory_space=pltpu.SMEM),
            pl.BlockSpec((_RB, F), row),
            pl.BlockSpec((_RB, L), lambda r: (r, 0)),
            pl.BlockSpec((1, F), full2),
            pl.BlockSpec((1, F), full2),
            pl.BlockSpec((1, F), full2),
            pl.BlockSpec((3, F), full2),
            pl.BlockSpec((1, F), full2),
            pl.BlockSpec((F, F), full2),
        ],
        out_specs=pl.BlockSpec((_RB, F), row),
        out_shape=jax.ShapeDtypeStruct((N, F), jnp.float32),
    )(hom3, agg, degp, b, g, bb, hw, hb, w2)


def _k6_body(hom_ref, agg_ref, degp_ref, b_ref, g_ref, bb_ref, hw_ref, hb_ref,
             muw_ref, muw1_ref, mub_ref, lvw_ref, lvw1_ref, lvb_ref,
             mu_ref, lv_ref):
    h, _ = _post_h(hom_ref, agg_ref, degp_ref, b_ref, g_ref, bb_ref,
                   hw_ref, hb_ref)
    mucst = (hom_ref[0] * muw1_ref[0:1, :] + hom_ref[1] * muw1_ref[1:2, :]
             + hom_ref[2] * muw1_ref[2:3, :] + mub_ref[...])
    lvcst = (hom_ref[0] * lvw1_ref[0:1, :] + hom_ref[1] * lvw1_ref[1:2, :]
             + hom_ref[2] * lvw1_ref[2:3, :] + lvb_ref[...])
    mu_ref[...] = jnp.dot(h, muw_ref[...],
                          preferred_element_type=jnp.float32) + mucst
    lv_ref[...] = jnp.dot(h, lvw_ref[...],
                          preferred_element_type=jnp.float32) + lvcst


def _k6_call(hom3, agg, degp, b, g, bb, hw, hb, muw, muw1, mub, lvw, lvw1, lvb):
    grid = (N // _RB,)
    row = lambda r: (r, 0)
    full2 = lambda r: (0, 0)
    return pl.pallas_call(
        _k6_body,
        grid=grid,
        in_specs=[
            pl.BlockSpec(memory_space=pltpu.SMEM),
            pl.BlockSpec((_RB, F), row),
            pl.BlockSpec((_RB, L), lambda r: (r, 0)),
            pl.BlockSpec((1, F), full2),
            pl.BlockSpec((1, F), full2),
            pl.BlockSpec((1, F), full2),
            pl.BlockSpec((3, F), full2),
            pl.BlockSpec((1, F), full2),
            pl.BlockSpec((F, LAT), full2),
            pl.BlockSpec((3, LAT), full2),
            pl.BlockSpec((1, LAT), full2),
            pl.BlockSpec((F, LAT), full2),
            pl.BlockSpec((3, LAT), full2),
            pl.BlockSpec((1, LAT), full2),
        ],
        out_specs=[pl.BlockSpec((_RB, LAT), row), pl.BlockSpec((_RB, LAT), row)],
        out_shape=[jax.ShapeDtypeStruct((N, LAT), jnp.float32),
                   jax.ShapeDtypeStruct((N, LAT), jnp.float32)],
    )(hom3, agg, degp, b, g, bb, hw, hb, muw, muw1, mub, lvw, lvw1, lvb)


_ARB = 512
_ACB = 2048


def _adj_body(zi_ref, zj_ref, out_ref):
    prod = lax.dot_general(zi_ref[...], zj_ref[...], (((1,), (1,)), ((), ())),
                           preferred_element_type=jnp.float32)
    out_ref[...] = jax.nn.sigmoid(prod)


def _adj_call(z):
    grid = (pl.cdiv(N, _ARB), pl.cdiv(N, _ACB))
    return pl.pallas_call(
        _adj_body,
        grid=grid,
        in_specs=[
            pl.BlockSpec((_ARB, LAT), lambda r, c: (r, 0)),
            pl.BlockSpec((_ACB, LAT), lambda r, c: (c, 0)),
        ],
        out_specs=pl.BlockSpec((_ARB, _ACB), lambda r, c: (r, c)),
        out_shape=jax.ShapeDtypeStruct((N, N), jnp.float32),
    )(z, z)


def _k8_body(z_ref, pw_ref, pb_ref, pg_ref, pbb_ref, t1w_ref, t1b_ref,
             t2w_ref, t2b_ref, l1w_ref, l1b_ref, l2w_ref, l2b_ref,
             xr_ref, yl_ref):
    z = z_ref[...]
    zp = (jnp.dot(z, pw_ref[...], preferred_element_type=jnp.float32)
          + pb_ref[...]) * _BN_S * pg_ref[...] + pbb_ref[...]
    t = jnp.maximum(jnp.dot(zp, t1w_ref[...],
                            preferred_element_type=jnp.float32) + t1b_ref[...], 0.0)
    xr_ref[...] = jnp.dot(t, t2w_ref[...],
                          preferred_element_type=jnp.float32) + t2b_ref[...]
    u = jnp.maximum(jnp.dot(z, l1w_ref[...],
                            preferred_element_type=jnp.float32) + l1b_ref[...], 0.0)
    yl_ref[...] = jnp.dot(u, l2w_ref[...],
                          preferred_element_type=jnp.float32) + l2b_ref[...]


def _k8_call(z, pw, pb, pg, pbb, t1w, t1b, t2w, t2b, l1w, l1b, l2w, l2b):
    grid = (N // _RB,)
    row = lambda r: (r, 0)
    full2 = lambda r: (0, 0)
    tl = pw.shape[1]
    return pl.pallas_call(
        _k8_body,
        grid=grid,
        in_specs=[
            pl.BlockSpec((_RB, LAT), row),
            pl.BlockSpec((LAT, tl), full2),
            pl.BlockSpec((1, tl), full2),
            pl.BlockSpec((1, tl), full2),
            pl.BlockSpec((1, tl), full2),
            pl.BlockSpec((tl, F), full2),
            pl.BlockSpec((1, F), full2),
            pl.BlockSpec((F, F), full2),
            pl.BlockSpec((1, F), full2),
            pl.BlockSpec((LAT, LAT), full2),
            pl.BlockSpec((1, LAT), full2),
            pl.BlockSpec((LAT, 128), full2),
            pl.BlockSpec((1, 128), full2),
        ],
        out_specs=[pl.BlockSpec((_RB, F), row), pl.BlockSpec((_RB, 128), row)],
        out_shape=[jax.ShapeDtypeStruct((N, F), jnp.float32),
                   jax.ShapeDtypeStruct((N, 128), jnp.float32)],
    )(z, pw, pb, pg, pbb, t1w, t1b, t2w, t2b, l1w, l1b, l2w, l2b)


def _k9_body(z_ref, lh1_ref, lh1b_ref, lh2_ref, lh2b_ref,
             sh1_ref, sh1b_ref, sh2_ref, sh2b_ref,
             fh1_ref, fh1b_ref, fh2_ref, fh2b_ref,
             lh_ref, sh_ref, fh_ref):
    zg = jnp.sum(z_ref[...], axis=0, keepdims=True) * (1.0 / N)

    def head(w1, b1, w2, b2):
        u = jnp.maximum(jnp.dot(zg, w1[...],
                                preferred_element_type=jnp.float32) + b1[...], 0.0)
        return jnp.dot(u, w2[...], preferred_element_type=jnp.float32) + b2[...]

    lh_ref[...] = jax.nn.sigmoid(head(lh1_ref, lh1b_ref, lh2_ref, lh2b_ref))
    sh_ref[...] = jax.nn.sigmoid(head(sh1_ref, sh1b_ref, sh2_ref, sh2b_ref))
    fh_ref[...] = jnp.tanh(head(fh1_ref, fh1b_ref, fh2_ref, fh2b_ref))


def _k9_call(z, args):
    specs = [pl.BlockSpec((N, LAT), lambda: (0, 0))]
    for a in args:
        specs.append(pl.BlockSpec(a.shape, lambda: (0, 0)))
    return pl.pallas_call(
        _k9_body,
        in_specs=specs,
        out_specs=[pl.BlockSpec((1, 128), lambda: (0, 0))] * 3,
        out_shape=[jax.ShapeDtypeStruct((1, 128), jnp.float32)] * 3,
    )(z, *args)


def kernel(x, edge_index, homophily_cond, batch, params):
    p = params
    e = edge_index.shape[1]
    loop = jnp.arange(N, dtype=jnp.int32)
    src = jnp.concatenate([edge_index[0].astype(jnp.int32), loop])
    dst = jnp.concatenate([edge_index[1].astype(jnp.int32), loop])
    ep = ((e + N + SUP - 1) // SUP) * SUP
    src = jnp.pad(src, (0, ep - e - N))
    dst = jnp.pad(dst, (0, ep - e - N), constant_values=N)

    zeros_deg = jnp.zeros((AROWS, L), jnp.float32)
    zeros_acc = jnp.zeros((AROWS, F), jnp.float32)
    hom3 = homophily_cond[0]

    r1 = lambda a: a.reshape(1, -1)

    degp, srcl, dstl, cntl = _deg_call(src, dst, zeros_deg)

    hw1 = _k2_call(x, p['gcn0_W'], degp)
    agg1 = _agg_call(hw1, srcl, dstl, cntl, zeros_acc)[:N]

    hw2 = _k4_call(hom3, agg1, degp, r1(p['gcn0_b']), r1(p['bn0_g']),
                   r1(p['bn0_b']), p['hom0_W'], r1(p['hom0_b']), p['gcn1_W'])
    agg2 = _agg_call(hw2, srcl, dstl, cntl, zeros_acc)[:N]

    muw0, muw1 = p['mu_W'][:F], p['mu_W'][F:]
    lvw0, lvw1 = p['lv_W'][:F], p['lv_W'][F:]
    mu, logvar = _k6_call(hom3, agg2, degp, r1(p['gcn1_b']), r1(p['bn1_g']),
                          r1(p['bn1_b']), p['hom1_W'], r1(p['hom1_b']),
                          muw0, muw1, r1(p['mu_b']), lvw0, lvw1, r1(p['lv_b']))
    z = mu

    adj = _adj_call(z)

    lab2w = jnp.pad(p['lab2_W'], ((0, 0), (0, 128 - NCLS)))
    lab2b = jnp.pad(r1(p['lab2_b']), ((0, 0), (0, 128 - NCLS)))
    x_recon, ylp = _k8_call(z, p['proj_W'], r1(p['proj_b']), r1(p['projbn_g']),
                            r1(p['projbn_b']), p['t1_W'], r1(p['t1_b']),
                            p['t2_W'], r1(p['t2_b']), p['lab1_W'],
                            r1(p['lab1_b']), lab2w, lab2b)
    y_logits = ylp[:, :NCLS]

    pad1 = lambda a: jnp.pad(r1(a), ((0, 0), (0, 128 - a.reshape(-1).shape[0])))
    head_args = (p['lh1_W'], r1(p['lh1_b']),
                 jnp.pad(p['lh2_W'], ((0, 0), (0, 127))), pad1(p['lh2_b']),
                 p['sh1_W'], r1(p['sh1_b']),
                 jnp.pad(p['sh2_W'], ((0, 0), (0, 127))), pad1(p['sh2_b']),
                 p['fh1_W'], r1(p['fh1_b']),
                 jnp.pad(p['fh2_W'], ((0, 0), (0, 127))), pad1(p['fh2_b']))
    lh, sh, fh = _k9_call(z, head_args)
    hom_pred = jnp.concatenate([lh[:, :1], sh[:, :1], fh[:, :1]], axis=1)

    return (adj, x_recon, y_logits, hom_pred, mu, logvar)


# speedup vs baseline: 3.4944x; 1.0068x over previous
import functools

import numpy as np
import jax
import jax.numpy as jnp
from jax import lax
from jax.experimental import pallas as pl
from jax.experimental.pallas import tpu as pltpu
from jax.experimental.pallas import tpu_sc as plsc

N = 10000
F = 256
LAT = 64
NCLS = 7

NC = 2
NS = 16
L = 16

_BN_S = float(1.0 / np.sqrt(1.0 + 1e-5))


NW = NC * NS
RPT = 320
AROWS = RPT + 8
NPAD = NW * RPT
ECHUNK = 128
SUP = 2688
MBUF = 288
def _cap(ep):
    return ep + ECHUNK

def _sc_mesh():
    return plsc.VectorSubcoreMesh(core_axis_name="c", subcore_axis_name="s")


def _scan_chunk(srcv, dstv, off, msrcv, mdstv, cnt, lo):
    lane = lax.iota(jnp.int32, L)
    for j in range(ECHUNK // L):
        d16 = dstv[pl.ds(off + j * L, L)]
        ld = d16 - lo
        ok = (ld >= 0) & (ld < RPT)
        inc = plsc.cumsum(ok.astype(jnp.int32))
        pos = jnp.where(ok, cnt + inc - 1, MBUF - L + lane)
        plsc.store_scatter(msrcv, [pos], srcv[pl.ds(off + j * L, L)])
        plsc.store_scatter(mdstv, [pos], ld)
        cnt = cnt + plsc.all_reduce_population_count(ok)
    return cnt


def _pad_tail(msrcv, mdstv, cnt):
    for g in range(ECHUNK // L):
        lane = lax.iota(jnp.int32, L) + g * L
        m = lane < cnt
        msrcv[pl.ds(g * L, L)] = jnp.where(m, msrcv[pl.ds(g * L, L)], 0)
        mdstv[pl.ds(g * L, L)] = jnp.where(m, mdstv[pl.ds(g * L, L)], RPT)


def _shift_tail(msrcv, mdstv):
    for g in range(L - ECHUNK // L):
        sl = pl.ds(ECHUNK + g * L, L)
        dl = pl.ds(g * L, L)
        msrcv[dl] = msrcv[sl]
        mdstv[dl] = mdstv[sl]


def _deg_flush(mdstv, acc):
    one0 = jnp.where(lax.iota(jnp.int32, L) == 0, 1.0, 0.0)
    cols = lax.iota(jnp.int32, L)

    @pl.loop(0, ECHUNK)
    def _edges(e):
        row = plsc.load_gather(mdstv, [jnp.full((L,), e, jnp.int32)])
        plsc.addupdate_scatter(acc, [row, cols], one0)


def _deg_body(dst_hbm, src_hbm, zeros_hbm, out_hbm, srcl_hbm, dstl_hbm,
              cntl_hbm, srcv, dstv, msrcv, mdstv, cntbuf, acc):
    c = lax.axis_index("c")
    s = lax.axis_index("s")
    wid = c * NS + s
    lo = wid * RPT
    pltpu.sync_copy(zeros_hbm, acc)
    ep = dst_hbm.shape[0]

    def _emit(nf):
        _deg_flush(mdstv, acc)
        pltpu.sync_copy(msrcv.at[pl.ds(0, ECHUNK)],
                        srcl_hbm.at[wid, pl.ds(nf * ECHUNK, ECHUNK)])
        pltpu.sync_copy(mdstv.at[pl.ds(0, ECHUNK)],
                        dstl_hbm.at[wid, pl.ds(nf * ECHUNK, ECHUNK)])

    @pl.loop(0, ep // SUP, init_carry=(jnp.zeros((L,), jnp.int32), 0))
    def _supers(i, carry):
        cnt0, nf0 = carry
        pltpu.sync_copy(src_hbm.at[pl.ds(i * SUP, SUP)], srcv)
        pltpu.sync_copy(dst_hbm.at[pl.ds(i * SUP, SUP)], dstv)

        @pl.loop(0, SUP // ECHUNK, init_carry=(cnt0, nf0))
        def _chunks(k, carry2):
            cnt, nf = carry2
            cnt = _scan_chunk(srcv, dstv, k * ECHUNK, msrcv, mdstv, cnt, lo)
            full = jnp.any(cnt >= ECHUNK)

            @pl.when(full)
            def _flush():
                _emit(nf)
                _shift_tail(msrcv, mdstv)

            return (jnp.where(full, cnt - ECHUNK, cnt),
                    jnp.where(full, nf + 1, nf))

        return _chunks

    cnt, nf = _supers
    _pad_tail(msrcv, mdstv, cnt)
    _emit(nf)
    cntbuf[...] = jnp.full((L,), (nf + 1) * ECHUNK, jnp.int32)
    pltpu.sync_copy(cntbuf, cntl_hbm.at[wid])
    pltpu.sync_copy(acc.at[pl.ds(0, RPT)], out_hbm.at[pl.ds(lo, RPT)])


def _deg_call(src_pad, dst_pad, zeros_deg):
    ep = src_pad.shape[0]
    cap = _cap(ep)
    return pl.kernel(
        _deg_body,
        out_type=[jax.ShapeDtypeStruct((NPAD, L), jnp.float32),
                  jax.ShapeDtypeStruct((NW, cap), jnp.int32),
                  jax.ShapeDtypeStruct((NW, cap), jnp.int32),
                  jax.ShapeDtypeStruct((NW, L), jnp.int32)],
        mesh=_sc_mesh(),
        compiler_params=pltpu.CompilerParams(needs_layout_passes=False),
        scratch_types=[
            pltpu.VMEM((SUP,), jnp.int32),
            pltpu.VMEM((SUP,), jnp.int32),
            pltpu.VMEM((MBUF,), jnp.int32),
            pltpu.VMEM((MBUF,), jnp.int32),
            pltpu.VMEM((L,), jnp.int32),
            pltpu.VMEM((AROWS, L), jnp.float32),
        ],
    )(dst_pad, src_pad, zeros_deg)


BBLK = 64


def _acc_block(mdstv, rows, acc):
    cols = lax.iota(jnp.int32, L)

    @pl.loop(0, BBLK)
    def _edges(e):
        row = plsc.load_gather(mdstv, [jnp.full((L,), e, jnp.int32)])
        for k in range(F // L):
            plsc.addupdate_scatter(acc, [row, cols + k * L],
                                   rows[e, pl.ds(k * L, L)])


def _agg_body(hw_hbm, srcl_hbm, dstl_hbm, cntl_hbm, zeros_hbm, out_hbm,
              cntv, mdstv, gidx, rows, acc, sem):
    c = lax.axis_index("c")
    s = lax.axis_index("s")
    wid = c * NS + s
    lo = wid * RPT
    pltpu.sync_copy(zeros_hbm, acc)
    pltpu.sync_copy(cntl_hbm.at[wid], cntv)
    cnt = cntv[...]
    cap = srcl_hbm.shape[1]
    nblk = cap // BBLK

    def _issue(i, b):
        pltpu.sync_copy(srcl_hbm.at[wid, pl.ds(i * BBLK, BBLK)], gidx.at[b])
        pltpu.sync_copy(dstl_hbm.at[wid, pl.ds(i * BBLK, BBLK)], mdstv.at[b])
        pltpu.async_copy(hw_hbm.at[gidx.at[b]], rows.at[b], sem.at[b])

    _issue(0, 0)

    @pl.loop(0, nblk, step=2)
    def _blocks(i0):
        for b in range(2):
            i = i0 + b
            nxt = (b + 1) % 2

            @pl.when(jnp.any((i + 1) * BBLK < cnt))
            def _prefetch():
                _issue(i + 1, nxt)

            @pl.when(jnp.any(i * BBLK < cnt))
            def _do():
                pltpu.make_async_copy(hw_hbm.at[gidx.at[b]], rows.at[b],
                                      sem.at[b]).wait()
                _acc_block(mdstv.at[b], rows.at[b], acc)

    pltpu.sync_copy(acc.at[pl.ds(0, RPT)], out_hbm.at[pl.ds(lo, RPT)])


def _agg_call(hw, srcl, dstl, cntl, zeros_acc):
    return pl.kernel(
        _agg_body,
        out_type=jax.ShapeDtypeStruct((NPAD, F), jnp.float32),
        mesh=_sc_mesh(),
        compiler_params=pltpu.CompilerParams(needs_layout_passes=False),
        scratch_types=[
            pltpu.VMEM((L,), jnp.int32),
            pltpu.VMEM((2, BBLK), jnp.int32),
            pltpu.VMEM((2, BBLK), jnp.int32),
            pltpu.VMEM((2, BBLK, F), jnp.float32),
            pltpu.VMEM((AROWS, F), jnp.float32),
            pltpu.SemaphoreType.DMA((2,)),
        ],
    )(hw, srcl, dstl, cntl, zeros_acc)


_RB = 1000


def _dis_of(degp_ref):
    deg = degp_ref[:, 0:1]
    return jnp.where(deg > 0.0, lax.rsqrt(deg), 0.0)


def _k2_body(x_ref, w_ref, degp_ref, out_ref):
    dis = _dis_of(degp_ref)
    out_ref[...] = jnp.dot(x_ref[...], w_ref[...],
                           preferred_element_type=jnp.float32) * dis


def _k2_call(x, w, degp):
    grid = (N // _RB,)
    return pl.pallas_call(
        _k2_body,
        grid=grid,
        in_specs=[
            pl.BlockSpec((_RB, F), lambda r: (r, 0)),
            pl.BlockSpec((F, F), lambda r: (0, 0)),
            pl.BlockSpec((_RB, L), lambda r: (r, 0)),
        ],
        out_specs=pl.BlockSpec((_RB, F), lambda r: (r, 0)),
        out_shape=jax.ShapeDtypeStruct((N, F), jnp.float32),
    )(x, w, degp)


def _post_h(hom_ref, agg_ref, degp_ref, b_ref, g_ref, bb_ref, hw_ref, hb_ref):
    dis = _dis_of(degp_ref)
    gcn = agg_ref[...] * dis + b_ref[...]
    h = jnp.maximum(gcn * _BN_S * g_ref[...] + bb_ref[...], 0.0)
    hv = (hom_ref[0] * hw_ref[0:1, :] + hom_ref[1] * hw_ref[1:2, :]
          + hom_ref[2] * hw_ref[2:3, :] + hb_ref[...])
    return h + hv, dis


def _k4_body(hom_ref, agg_ref, degp_ref, b_ref, g_ref, bb_ref, hw_ref, hb_ref,
             w2_ref, out_ref):
    h, dis = _post_h(hom_ref, agg_ref, degp_ref, b_ref, g_ref, bb_ref,
                     hw_ref, hb_ref)
    out_ref[...] = jnp.dot(h, w2_ref[...],
                           preferred_element_type=jnp.float32) * dis


def _k4_call(hom3, agg, degp, b, g, bb, hw, hb, w2):
    grid = (N // _RB,)
    row = lambda r: (r, 0)
    full2 = lambda r: (0, 0)
    return pl.pallas_call(
        _k4_body,
        grid=grid,
        in_specs=[
            pl.BlockSpec(memory_space=pltpu.SMEM),
            pl.BlockSpec((_RB, F), row),
            pl.BlockSpec((_RB, L), lambda r: (r, 0)),
            pl.BlockSpec((1, F), full2),
            pl.BlockSpec((1, F), full2),
            pl.BlockSpec((1, F), full2),
            pl.BlockSpec((3, F), full2),
            pl.BlockSpec((1, F), full2),
            pl.BlockSpec((F, F), full2),
        ],
        out_specs=pl.BlockSpec((_RB, F), row),
        out_shape=jax.ShapeDtypeStruct((N, F), jnp.float32),
    )(hom3, agg, degp, b, g, bb, hw, hb, w2)


def _k6_body(hom_ref, agg_ref, degp_ref, b_ref, g_ref, bb_ref, hw_ref, hb_ref,
             muw_ref, muw1_ref, mub_ref, lvw_ref, lvw1_ref, lvb_ref,
             mu_ref, lv_ref):
    h, _ = _post_h(hom_ref, agg_ref, degp_ref, b_ref, g_ref, bb_ref,
                   hw_ref, hb_ref)
    mucst = (hom_ref[0] * muw1_ref[0:1, :] + hom_ref[1] * muw1_ref[1:2, :]
             + hom_ref[2] * muw1_ref[2:3, :] + mub_ref[...])
    lvcst = (hom_ref[0] * lvw1_ref[0:1, :] + hom_ref[1] * lvw1_ref[1:2, :]
             + hom_ref[2] * lvw1_ref[2:3, :] + lvb_ref[...])
    mu_ref[...] = jnp.dot(h, muw_ref[...],
                          preferred_element_type=jnp.float32) + mucst
    lv_ref[...] = jnp.dot(h, lvw_ref[...],
                          preferred_element_type=jnp.float32) + lvcst


def _k6_call(hom3, agg, degp, b, g, bb, hw, hb, muw, muw1, mub, lvw, lvw1, lvb):
    grid = (N // _RB,)
    row = lambda r: (r, 0)
    full2 = lambda r: (0, 0)
    return pl.pallas_call(
        _k6_body,
        grid=grid,
        in_specs=[
            pl.BlockSpec(memory_space=pltpu.SMEM),
            pl.BlockSpec((_RB, F), row),
            pl.BlockSpec((_RB, L), lambda r: (r, 0)),
            pl.BlockSpec((1, F), full2),
            pl.BlockSpec((1, F), full2),
            pl.BlockSpec((1, F), full2),
            pl.BlockSpec((3, F), full2),
            pl.BlockSpec((1, F), full2),
            pl.BlockSpec((F, LAT), full2),
            pl.BlockSpec((3, LAT), full2),
            pl.BlockSpec((1, LAT), full2),
            pl.BlockSpec((F, LAT), full2),
            pl.BlockSpec((3, LAT), full2),
            pl.BlockSpec((1, LAT), full2),
        ],
        out_specs=[pl.BlockSpec((_RB, LAT), row), pl.BlockSpec((_RB, LAT), row)],
        out_shape=[jax.ShapeDtypeStruct((N, LAT), jnp.float32),
                   jax.ShapeDtypeStruct((N, LAT), jnp.float32)],
    )(hom3, agg, degp, b, g, bb, hw, hb, muw, muw1, mub, lvw, lvw1, lvb)


_ARB = 512
_ACB = 2048


def _adj_body(zi_ref, zj_ref, out_ref):
    prod = lax.dot_general(zi_ref[...], zj_ref[...], (((1,), (1,)), ((), ())),
                           preferred_element_type=jnp.float32)
    out_ref[...] = jax.nn.sigmoid(prod)


def _adj_call(z):
    grid = (pl.cdiv(N, _ARB), pl.cdiv(N, _ACB))
    return pl.pallas_call(
        _adj_body,
        grid=grid,
        in_specs=[
            pl.BlockSpec((_ARB, LAT), lambda r, c: (r, 0)),
            pl.BlockSpec((_ACB, LAT), lambda r, c: (c, 0)),
        ],
        out_specs=pl.BlockSpec((_ARB, _ACB), lambda r, c: (r, c)),
        out_shape=jax.ShapeDtypeStruct((N, N), jnp.float32),
    )(z, z)


def _k8_body(z_ref, pw_ref, pb_ref, pg_ref, pbb_ref, t1w_ref, t1b_ref,
             t2w_ref, t2b_ref, l1w_ref, l1b_ref, l2w_ref, l2b_ref,
             xr_ref, yl_ref):
    z = z_ref[...]
    zp = (jnp.dot(z, pw_ref[...], preferred_element_type=jnp.float32)
          + pb_ref[...]) * _BN_S * pg_ref[...] + pbb_ref[...]
    t = jnp.maximum(jnp.dot(zp, t1w_ref[...],
                            preferred_element_type=jnp.float32) + t1b_ref[...], 0.0)
    xr_ref[...] = jnp.dot(t, t2w_ref[...],
                          preferred_element_type=jnp.float32) + t2b_ref[...]
    u = jnp.maximum(jnp.dot(z, l1w_ref[...],
                            preferred_element_type=jnp.float32) + l1b_ref[...], 0.0)
    yl_ref[...] = jnp.dot(u, l2w_ref[...],
                          preferred_element_type=jnp.float32) + l2b_ref[...]


def _k8_call(z, pw, pb, pg, pbb, t1w, t1b, t2w, t2b, l1w, l1b, l2w, l2b):
    grid = (N // _RB,)
    row = lambda r: (r, 0)
    full2 = lambda r: (0, 0)
    tl = pw.shape[1]
    return pl.pallas_call(
        _k8_body,
        grid=grid,
        in_specs=[
            pl.BlockSpec((_RB, LAT), row),
            pl.BlockSpec((LAT, tl), full2),
            pl.BlockSpec((1, tl), full2),
            pl.BlockSpec((1, tl), full2),
            pl.BlockSpec((1, tl), full2),
            pl.BlockSpec((tl, F), full2),
            pl.BlockSpec((1, F), full2),
            pl.BlockSpec((F, F), full2),
            pl.BlockSpec((1, F), full2),
            pl.BlockSpec((LAT, LAT), full2),
            pl.BlockSpec((1, LAT), full2),
            pl.BlockSpec((LAT, 128), full2),
            pl.BlockSpec((1, 128), full2),
        ],
        out_specs=[pl.BlockSpec((_RB, F), row), pl.BlockSpec((_RB, 128), row)],
        out_shape=[jax.ShapeDtypeStruct((N, F), jnp.float32),
                   jax.ShapeDtypeStruct((N, 128), jnp.float32)],
    )(z, pw, pb, pg, pbb, t1w, t1b, t2w, t2b, l1w, l1b, l2w, l2b)


def _k9_body(z_ref, lh1_ref, lh1b_ref, lh2_ref, lh2b_ref,
             sh1_ref, sh1b_ref, sh2_ref, sh2b_ref,
             fh1_ref, fh1b_ref, fh2_ref, fh2b_ref,
             lh_ref, sh_ref, fh_ref):
    zg = jnp.sum(z_ref[...], axis=0, keepdims=True) * (1.0 / N)

    def head(w1, b1, w2, b2):
        u = jnp.maximum(jnp.dot(zg, w1[...],
                                preferred_element_type=jnp.float32) + b1[...], 0.0)
        return jnp.dot(u, w2[...], preferred_element_type=jnp.float32) + b2[...]

    lh_ref[...] = jax.nn.sigmoid(head(lh1_ref, lh1b_ref, lh2_ref, lh2b_ref))
    sh_ref[...] = jax.nn.sigmoid(head(sh1_ref, sh1b_ref, sh2_ref, sh2b_ref))
    fh_ref[...] = jnp.tanh(head(fh1_ref, fh1b_ref, fh2_ref, fh2b_ref))


def _k9_call(z, args):
    specs = [pl.BlockSpec((N, LAT), lambda: (0, 0))]
    for a in args:
        specs.append(pl.BlockSpec(a.shape, lambda: (0, 0)))
    return pl.pallas_call(
        _k9_body,
        in_specs=specs,
        out_specs=[pl.BlockSpec((1, 128), lambda: (0, 0))] * 3,
        out_shape=[jax.ShapeDtypeStruct((1, 128), jnp.float32)] * 3,
    )(z, *args)


def kernel(x, edge_index, homophily_cond, batch, params):
    p = params
    e = edge_index.shape[1]
    loop = jnp.arange(N, dtype=jnp.int32)
    src = jnp.concatenate([edge_index[0].astype(jnp.int32), loop])
    dst = jnp.concatenate([edge_index[1].astype(jnp.int32), loop])
    ep = ((e + N + SUP - 1) // SUP) * SUP
    src = jnp.pad(src, (0, ep - e - N))
    dst = jnp.pad(dst, (0, ep - e - N), constant_values=N)

    zeros_deg = jnp.zeros((AROWS, L), jnp.float32)
    zeros_acc = jnp.zeros((AROWS, F), jnp.float32)
    hom3 = homophily_cond[0]

    r1 = lambda a: a.reshape(1, -1)

    degp, srcl, dstl, cntl = _deg_call(src, dst, zeros_deg)

    hw1 = _k2_call(x, p['gcn0_W'], degp)
    agg1 = _agg_call(hw1, srcl, dstl, cntl, zeros_acc)[:N]

    hw2 = _k4_call(hom3, agg1, degp, r1(p['gcn0_b']), r1(p['bn0_g']),
                   r1(p['bn0_b']), p['hom0_W'], r1(p['hom0_b']), p['gcn1_W'])
    agg2 = _agg_call(hw2, srcl, dstl, cntl, zeros_acc)[:N]

    muw0, muw1 = p['mu_W'][:F], p['mu_W'][F:]
    lvw0, lvw1 = p['lv_W'][:F], p['lv_W'][F:]
    mu, logvar = _k6_call(hom3, agg2, degp, r1(p['gcn1_b']), r1(p['bn1_g']),
                          r1(p['bn1_b']), p['hom1_W'], r1(p['hom1_b']),
                          muw0, muw1, r1(p['mu_b']), lvw0, lvw1, r1(p['lv_b']))
    z = mu

    adj = _adj_call(z)

    lab2w = jnp.pad(p['lab2_W'], ((0, 0), (0, 128 - NCLS)))
    lab2b = jnp.pad(r1(p['lab2_b']), ((0, 0), (0, 128 - NCLS)))
    x_recon, ylp = _k8_call(z, p['proj_W'], r1(p['proj_b']), r1(p['projbn_g']),
                            r1(p['projbn_b']), p['t1_W'], r1(p['t1_b']),
                            p['t2_W'], r1(p['t2_b']), p['lab1_W'],
                            r1(p['lab1_b']), lab2w, lab2b)
    y_logits = ylp[:, :NCLS]

    pad1 = lambda a: jnp.pad(r1(a), ((0, 0), (0, 128 - a.reshape(-1).shape[0])))
    head_args = (p['lh1_W'], r1(p['lh1_b']),
                 jnp.pad(p['lh2_W'], ((0, 0), (0, 127))), pad1(p['lh2_b']),
                 p['sh1_W'], r1(p['sh1_b']),
                 jnp.pad(p['sh2_W'], ((0, 0), (0, 127))), pad1(p['sh2_b']),
                 p['fh1_W'], r1(p['fh1_b']),
                 jnp.pad(p['fh2_W'], ((0, 0), (0, 127))), pad1(p['fh2_b']))
    lh, sh, fh = _k9_call(z, head_args)
    hom_pred = jnp.concatenate([lh[:, :1], sh[:, :1], fh[:, :1]], axis=1)

    return (adj, x_recon, y_logits, hom_pred, mu, logvar)


# speedup vs baseline: 4.5590x; 1.3047x over previous
import functools

import numpy as np
import jax
import jax.numpy as jnp
from jax import lax
from jax.experimental import pallas as pl
from jax.experimental.pallas import tpu as pltpu
from jax.experimental.pallas import tpu_sc as plsc

N = 10000
F = 256
LAT = 64
NCLS = 7

NC = 2
NS = 16
L = 16

_BN_S = float(1.0 / np.sqrt(1.0 + 1e-5))


NW = NC * NS
RPT = 320
AROWS = RPT + 8
NPAD = NW * RPT
ECHUNK = 128
SUP = 2688
MBUF = 288
def _cap(ep):
    return ep + ECHUNK

def _sc_mesh():
    return plsc.VectorSubcoreMesh(core_axis_name="c", subcore_axis_name="s")


def _scan_chunk(srcv, dstv, off, msrcv, mdstv, cnt, lo):
    lane = lax.iota(jnp.int32, L)
    for j in range(ECHUNK // L):
        d16 = dstv[pl.ds(off + j * L, L)]
        ld = d16 - lo
        ok = (ld >= 0) & (ld < RPT)
        inc = plsc.cumsum(ok.astype(jnp.int32))
        pos = jnp.where(ok, cnt + inc - 1, MBUF - L + lane)
        plsc.store_scatter(msrcv, [pos], srcv[pl.ds(off + j * L, L)])
        plsc.store_scatter(mdstv, [pos], ld)
        cnt = cnt + plsc.all_reduce_population_count(ok)
    return cnt


def _pad_tail(msrcv, mdstv, cnt):
    for g in range(ECHUNK // L):
        lane = lax.iota(jnp.int32, L) + g * L
        m = lane < cnt
        msrcv[pl.ds(g * L, L)] = jnp.where(m, msrcv[pl.ds(g * L, L)], 0)
        mdstv[pl.ds(g * L, L)] = jnp.where(m, mdstv[pl.ds(g * L, L)], RPT)


def _shift_tail(msrcv, mdstv):
    for g in range(L - ECHUNK // L):
        sl = pl.ds(ECHUNK + g * L, L)
        dl = pl.ds(g * L, L)
        msrcv[dl] = msrcv[sl]
        mdstv[dl] = mdstv[sl]


def _deg_flush(mdstv, acc):
    one0 = jnp.where(lax.iota(jnp.int32, L) == 0, 1.0, 0.0)
    cols = lax.iota(jnp.int32, L)

    @pl.loop(0, ECHUNK)
    def _edges(e):
        row = plsc.load_gather(mdstv, [jnp.full((L,), e, jnp.int32)])
        plsc.addupdate_scatter(acc, [row, cols], one0)


def _deg_body(dst_hbm, src_hbm, zeros_hbm, out_hbm, srcl_hbm, dstl_hbm,
              cntl_hbm, srcv, dstv, msrcv, mdstv, cntbuf, acc):
    c = lax.axis_index("c")
    s = lax.axis_index("s")
    wid = c * NS + s
    lo = wid * RPT
    pltpu.sync_copy(zeros_hbm, acc)
    ep = dst_hbm.shape[0]

    def _emit(nf):
        _deg_flush(mdstv, acc)
        pltpu.sync_copy(msrcv.at[pl.ds(0, ECHUNK)],
                        srcl_hbm.at[wid, pl.ds(nf * ECHUNK, ECHUNK)])
        pltpu.sync_copy(mdstv.at[pl.ds(0, ECHUNK)],
                        dstl_hbm.at[wid, pl.ds(nf * ECHUNK, ECHUNK)])

    @pl.loop(0, ep // SUP, init_carry=(jnp.zeros((L,), jnp.int32), 0))
    def _supers(i, carry):
        cnt0, nf0 = carry
        pltpu.sync_copy(src_hbm.at[pl.ds(i * SUP, SUP)], srcv)
        pltpu.sync_copy(dst_hbm.at[pl.ds(i * SUP, SUP)], dstv)

        @pl.loop(0, SUP // ECHUNK, init_carry=(cnt0, nf0))
        def _chunks(k, carry2):
            cnt, nf = carry2
            cnt = _scan_chunk(srcv, dstv, k * ECHUNK, msrcv, mdstv, cnt, lo)
            full = jnp.any(cnt >= ECHUNK)

            @pl.when(full)
            def _flush():
                _emit(nf)
                _shift_tail(msrcv, mdstv)

            return (jnp.where(full, cnt - ECHUNK, cnt),
                    jnp.where(full, nf + 1, nf))

        return _chunks

    cnt, nf = _supers
    _pad_tail(msrcv, mdstv, cnt)
    _emit(nf)
    cntbuf[...] = jnp.full((L,), (nf + 1) * ECHUNK, jnp.int32)
    pltpu.sync_copy(cntbuf, cntl_hbm.at[wid])
    pltpu.sync_copy(acc.at[pl.ds(0, RPT)], out_hbm.at[pl.ds(lo, RPT)])


def _deg_call(src_pad, dst_pad, zeros_deg):
    ep = src_pad.shape[0]
    cap = _cap(ep)
    return pl.kernel(
        _deg_body,
        out_type=[jax.ShapeDtypeStruct((NPAD, L), jnp.float32),
                  jax.ShapeDtypeStruct((NW, cap), jnp.int32),
                  jax.ShapeDtypeStruct((NW, cap), jnp.int32),
                  jax.ShapeDtypeStruct((NW, L), jnp.int32)],
        mesh=_sc_mesh(),
        compiler_params=pltpu.CompilerParams(needs_layout_passes=False),
        scratch_types=[
            pltpu.VMEM((SUP,), jnp.int32),
            pltpu.VMEM((SUP,), jnp.int32),
            pltpu.VMEM((MBUF,), jnp.int32),
            pltpu.VMEM((MBUF,), jnp.int32),
            pltpu.VMEM((L,), jnp.int32),
            pltpu.VMEM((AROWS, L), jnp.float32),
        ],
    )(dst_pad, src_pad, zeros_deg)


BBLK = 64


def _acc_block(mdstv, rows, acc):
    cols = lax.iota(jnp.int32, L)

    @plsc.parallel_loop(0, BBLK, unroll=4)
    def _edges(e):
        row = plsc.load_gather(mdstv, [jnp.full((L,), e, jnp.int32)])
        for k in range(F // L):
            plsc.addupdate_scatter(acc, [row, cols + k * L],
                                   rows[e, pl.ds(k * L, L)])


def _agg_body(hw_hbm, srcl_hbm, dstl_hbm, cntl_hbm, zeros_hbm, out_hbm,
              cntv, mdstv, gidx, rows, acc, sem):
    c = lax.axis_index("c")
    s = lax.axis_index("s")
    wid = c * NS + s
    lo = wid * RPT
    pltpu.sync_copy(zeros_hbm, acc)
    pltpu.sync_copy(cntl_hbm.at[wid], cntv)
    cnt = cntv[...]
    cap = srcl_hbm.shape[1]
    nblk = cap // BBLK

    def _issue(i, b):
        pltpu.sync_copy(srcl_hbm.at[wid, pl.ds(i * BBLK, BBLK)], gidx.at[b])
        pltpu.sync_copy(dstl_hbm.at[wid, pl.ds(i * BBLK, BBLK)], mdstv.at[b])
        pltpu.async_copy(hw_hbm.at[gidx.at[b]], rows.at[b], sem.at[b])

    _issue(0, 0)

    @pl.loop(0, nblk, step=2)
    def _blocks(i0):
        for b in range(2):
            i = i0 + b
            nxt = (b + 1) % 2

            @pl.when(jnp.any((i + 1) * BBLK < cnt))
            def _prefetch():
                _issue(i + 1, nxt)

            @pl.when(jnp.any(i * BBLK < cnt))
            def _do():
                pltpu.make_async_copy(hw_hbm.at[gidx.at[b]], rows.at[b],
                                      sem.at[b]).wait()
                _acc_block(mdstv.at[b], rows.at[b], acc)

    pltpu.sync_copy(acc.at[pl.ds(0, RPT)], out_hbm.at[pl.ds(lo, RPT)])


def _agg_call(hw, srcl, dstl, cntl, zeros_acc):
    return pl.kernel(
        _agg_body,
        out_type=jax.ShapeDtypeStruct((NPAD, F), jnp.float32),
        mesh=_sc_mesh(),
        compiler_params=pltpu.CompilerParams(needs_layout_passes=False),
        scratch_types=[
            pltpu.VMEM((L,), jnp.int32),
            pltpu.VMEM((2, BBLK), jnp.int32),
            pltpu.VMEM((2, BBLK), jnp.int32),
            pltpu.VMEM((2, BBLK, F), jnp.float32),
            pltpu.VMEM((AROWS, F), jnp.float32),
            pltpu.SemaphoreType.DMA((2,)),
        ],
    )(hw, srcl, dstl, cntl, zeros_acc)


_RB = 1000


def _dis_of(degp_ref):
    deg = degp_ref[:, 0:1]
    return jnp.where(deg > 0.0, lax.rsqrt(deg), 0.0)


def _k2_body(x_ref, w_ref, degp_ref, out_ref):
    dis = _dis_of(degp_ref)
    out_ref[...] = jnp.dot(x_ref[...], w_ref[...],
                           preferred_element_type=jnp.float32) * dis


def _k2_call(x, w, degp):
    grid = (N // _RB,)
    return pl.pallas_call(
        _k2_body,
        grid=grid,
        in_specs=[
            pl.BlockSpec((_RB, F), lambda r: (r, 0)),
            pl.BlockSpec((F, F), lambda r: (0, 0)),
            pl.BlockSpec((_RB, L), lambda r: (r, 0)),
        ],
        out_specs=pl.BlockSpec((_RB, F), lambda r: (r, 0)),
        out_shape=jax.ShapeDtypeStruct((N, F), jnp.float32),
    )(x, w, degp)


def _post_h(hom_ref, agg_ref, degp_ref, b_ref, g_ref, bb_ref, hw_ref, hb_ref):
    dis = _dis_of(degp_ref)
    gcn = agg_ref[...] * dis + b_ref[...]
    h = jnp.maximum(gcn * _BN_S * g_ref[...] + bb_ref[...], 0.0)
    hv = (hom_ref[0] * hw_ref[0:1, :] + hom_ref[1] * hw_ref[1:2, :]
          + hom_ref[2] * hw_ref[2:3, :] + hb_ref[...])
    return h + hv, dis


def _k4_body(hom_ref, agg_ref, degp_ref, b_ref, g_ref, bb_ref, hw_ref, hb_ref,
             w2_ref, out_ref):
    h, dis = _post_h(hom_ref, agg_ref, degp_ref, b_ref, g_ref, bb_ref,
                     hw_ref, hb_ref)
    out_ref[...] = jnp.dot(h, w2_ref[...],
                           preferred_element_type=jnp.float32) * dis


def _k4_call(hom3, agg, degp, b, g, bb, hw, hb, w2):
    grid = (N // _RB,)
    row = lambda r: (r, 0)
    full2 = lambda r: (0, 0)
    return pl.pallas_call(
        _k4_body,
        grid=grid,
        in_specs=[
            pl.BlockSpec(memory_space=pltpu.SMEM),
            pl.BlockSpec((_RB, F), row),
            pl.BlockSpec((_RB, L), lambda r: (r, 0)),
            pl.BlockSpec((1, F), full2),
            pl.BlockSpec((1, F), full2),
            pl.BlockSpec((1, F), full2),
            pl.BlockSpec((3, F), full2),
            pl.BlockSpec((1, F), full2),
            pl.BlockSpec((F, F), full2),
        ],
        out_specs=pl.BlockSpec((_RB, F), row),
        out_shape=jax.ShapeDtypeStruct((N, F), jnp.float32),
    )(hom3, agg, degp, b, g, bb, hw, hb, w2)


def _k6_body(hom_ref, agg_ref, degp_ref, b_ref, g_ref, bb_ref, hw_ref, hb_ref,
             muw_ref, muw1_ref, mub_ref, lvw_ref, lvw1_ref, lvb_ref,
             mu_ref, lv_ref):
    h, _ = _post_h(hom_ref, agg_ref, degp_ref, b_ref, g_ref, bb_ref,
                   hw_ref, hb_ref)
    mucst = (hom_ref[0] * muw1_ref[0:1, :] + hom_ref[1] * muw1_ref[1:2, :]
             + hom_ref[2] * muw1_ref[2:3, :] + mub_ref[...])
    lvcst = (hom_ref[0] * lvw1_ref[0:1, :] + hom_ref[1] * lvw1_ref[1:2, :]
             + hom_ref[2] * lvw1_ref[2:3, :] + lvb_ref[...])
    mu_ref[...] = jnp.dot(h, muw_ref[...],
                          preferred_element_type=jnp.float32) + mucst
    lv_ref[...] = jnp.dot(h, lvw_ref[...],
                          preferred_element_type=jnp.float32) + lvcst


def _k6_call(hom3, agg, degp, b, g, bb, hw, hb, muw, muw1, mub, lvw, lvw1, lvb):
    grid = (N // _RB,)
    row = lambda r: (r, 0)
    full2 = lambda r: (0, 0)
    return pl.pallas_call(
        _k6_body,
        grid=grid,
        in_specs=[
            pl.BlockSpec(memory_space=pltpu.SMEM),
            pl.BlockSpec((_RB, F), row),
            pl.BlockSpec((_RB, L), lambda r: (r, 0)),
            pl.BlockSpec((1, F), full2),
            pl.BlockSpec((1, F), full2),
            pl.BlockSpec((1, F), full2),
            pl.BlockSpec((3, F), full2),
            pl.BlockSpec((1, F), full2),
            pl.BlockSpec((F, LAT), full2),
            pl.BlockSpec((3, LAT), full2),
            pl.BlockSpec((1, LAT), full2),
            pl.BlockSpec((F, LAT), full2),
            pl.BlockSpec((3, LAT), full2),
            pl.BlockSpec((1, LAT), full2),
        ],
        out_specs=[pl.BlockSpec((_RB, LAT), row), pl.BlockSpec((_RB, LAT), row)],
        out_shape=[jax.ShapeDtypeStruct((N, LAT), jnp.float32),
                   jax.ShapeDtypeStruct((N, LAT), jnp.float32)],
    )(hom3, agg, degp, b, g, bb, hw, hb, muw, muw1, mub, lvw, lvw1, lvb)


_ARB = 512
_ACB = 2048


def _adj_body(zi_ref, zj_ref, out_ref):
    prod = lax.dot_general(zi_ref[...], zj_ref[...], (((1,), (1,)), ((), ())),
                           preferred_element_type=jnp.float32)
    out_ref[...] = jax.nn.sigmoid(prod)


def _adj_call(z):
    grid = (pl.cdiv(N, _ARB), pl.cdiv(N, _ACB))
    return pl.pallas_call(
        _adj_body,
        grid=grid,
        in_specs=[
            pl.BlockSpec((_ARB, LAT), lambda r, c: (r, 0)),
            pl.BlockSpec((_ACB, LAT), lambda r, c: (c, 0)),
        ],
        out_specs=pl.BlockSpec((_ARB, _ACB), lambda r, c: (r, c)),
        out_shape=jax.ShapeDtypeStruct((N, N), jnp.float32),
    )(z, z)


def _k8_body(z_ref, pw_ref, pb_ref, pg_ref, pbb_ref, t1w_ref, t1b_ref,
             t2w_ref, t2b_ref, l1w_ref, l1b_ref, l2w_ref, l2b_ref,
             xr_ref, yl_ref):
    z = z_ref[...]
    zp = (jnp.dot(z, pw_ref[...], preferred_element_type=jnp.float32)
          + pb_ref[...]) * _BN_S * pg_ref[...] + pbb_ref[...]
    t = jnp.maximum(jnp.dot(zp, t1w_ref[...],
                            preferred_element_type=jnp.float32) + t1b_ref[...], 0.0)
    xr_ref[...] = jnp.dot(t, t2w_ref[...],
                          preferred_element_type=jnp.float32) + t2b_ref[...]
    u = jnp.maximum(jnp.dot(z, l1w_ref[...],
                            preferred_element_type=jnp.float32) + l1b_ref[...], 0.0)
    yl_ref[...] = jnp.dot(u, l2w_ref[...],
                          preferred_element_type=jnp.float32) + l2b_ref[...]


def _k8_call(z, pw, pb, pg, pbb, t1w, t1b, t2w, t2b, l1w, l1b, l2w, l2b):
    grid = (N // _RB,)
    row = lambda r: (r, 0)
    full2 = lambda r: (0, 0)
    tl = pw.shape[1]
    return pl.pallas_call(
        _k8_body,
        grid=grid,
        in_specs=[
            pl.BlockSpec((_RB, LAT), row),
            pl.BlockSpec((LAT, tl), full2),
            pl.BlockSpec((1, tl), full2),
            pl.BlockSpec((1, tl), full2),
            pl.BlockSpec((1, tl), full2),
            pl.BlockSpec((tl, F), full2),
            pl.BlockSpec((1, F), full2),
            pl.BlockSpec((F, F), full2),
            pl.BlockSpec((1, F), full2),
            pl.BlockSpec((LAT, LAT), full2),
            pl.BlockSpec((1, LAT), full2),
            pl.BlockSpec((LAT, 128), full2),
            pl.BlockSpec((1, 128), full2),
        ],
        out_specs=[pl.BlockSpec((_RB, F), row), pl.BlockSpec((_RB, 128), row)],
        out_shape=[jax.ShapeDtypeStruct((N, F), jnp.float32),
                   jax.ShapeDtypeStruct((N, 128), jnp.float32)],
    )(z, pw, pb, pg, pbb, t1w, t1b, t2w, t2b, l1w, l1b, l2w, l2b)


def _k9_body(z_ref, lh1_ref, lh1b_ref, lh2_ref, lh2b_ref,
             sh1_ref, sh1b_ref, sh2_ref, sh2b_ref,
             fh1_ref, fh1b_ref, fh2_ref, fh2b_ref,
             lh_ref, sh_ref, fh_ref):
    zg = jnp.sum(z_ref[...], axis=0, keepdims=True) * (1.0 / N)

    def head(w1, b1, w2, b2):
        u = jnp.maximum(jnp.dot(zg, w1[...],
                                preferred_element_type=jnp.float32) + b1[...], 0.0)
        return jnp.dot(u, w2[...], preferred_element_type=jnp.float32) + b2[...]

    lh_ref[...] = jax.nn.sigmoid(head(lh1_ref, lh1b_ref, lh2_ref, lh2b_ref))
    sh_ref[...] = jax.nn.sigmoid(head(sh1_ref, sh1b_ref, sh2_ref, sh2b_ref))
    fh_ref[...] = jnp.tanh(head(fh1_ref, fh1b_ref, fh2_ref, fh2b_ref))


def _k9_call(z, args):
    specs = [pl.BlockSpec((N, LAT), lambda: (0, 0))]
    for a in args:
        specs.append(pl.BlockSpec(a.shape, lambda: (0, 0)))
    return pl.pallas_call(
        _k9_body,
        in_specs=specs,
        out_specs=[pl.BlockSpec((1, 128), lambda: (0, 0))] * 3,
        out_shape=[jax.ShapeDtypeStruct((1, 128), jnp.float32)] * 3,
    )(z, *args)


def kernel(x, edge_index, homophily_cond, batch, params):
    p = params
    e = edge_index.shape[1]
    loop = jnp.arange(N, dtype=jnp.int32)
    src = jnp.concatenate([edge_index[0].astype(jnp.int32), loop])
    dst = jnp.concatenate([edge_index[1].astype(jnp.int32), loop])
    ep = ((e + N + SUP - 1) // SUP) * SUP
    src = jnp.pad(src, (0, ep - e - N))
    dst = jnp.pad(dst, (0, ep - e - N), constant_values=N)

    zeros_deg = jnp.zeros((AROWS, L), jnp.float32)
    zeros_acc = jnp.zeros((AROWS, F), jnp.float32)
    hom3 = homophily_cond[0]

    r1 = lambda a: a.reshape(1, -1)

    degp, srcl, dstl, cntl = _deg_call(src, dst, zeros_deg)

    hw1 = _k2_call(x, p['gcn0_W'], degp)
    agg1 = _agg_call(hw1, srcl, dstl, cntl, zeros_acc)[:N]

    hw2 = _k4_call(hom3, agg1, degp, r1(p['gcn0_b']), r1(p['bn0_g']),
                   r1(p['bn0_b']), p['hom0_W'], r1(p['hom0_b']), p['gcn1_W'])
    agg2 = _agg_call(hw2, srcl, dstl, cntl, zeros_acc)[:N]

    muw0, muw1 = p['mu_W'][:F], p['mu_W'][F:]
    lvw0, lvw1 = p['lv_W'][:F], p['lv_W'][F:]
    mu, logvar = _k6_call(hom3, agg2, degp, r1(p['gcn1_b']), r1(p['bn1_g']),
                          r1(p['bn1_b']), p['hom1_W'], r1(p['hom1_b']),
                          muw0, muw1, r1(p['mu_b']), lvw0, lvw1, r1(p['lv_b']))
    z = mu

    adj = _adj_call(z)

    lab2w = jnp.pad(p['lab2_W'], ((0, 0), (0, 128 - NCLS)))
    lab2b = jnp.pad(r1(p['lab2_b']), ((0, 0), (0, 128 - NCLS)))
    x_recon, ylp = _k8_call(z, p['proj_W'], r1(p['proj_b']), r1(p['projbn_g']),
                            r1(p['projbn_b']), p['t1_W'], r1(p['t1_b']),
                            p['t2_W'], r1(p['t2_b']), p['lab1_W'],
                            r1(p['lab1_b']), lab2w, lab2b)
    y_logits = ylp[:, :NCLS]

    pad1 = lambda a: jnp.pad(r1(a), ((0, 0), (0, 128 - a.reshape(-1).shape[0])))
    head_args = (p['lh1_W'], r1(p['lh1_b']),
                 jnp.pad(p['lh2_W'], ((0, 0), (0, 127))), pad1(p['lh2_b']),
                 p['sh1_W'], r1(p['sh1_b']),
                 jnp.pad(p['sh2_W'], ((0, 0), (0, 127))), pad1(p['sh2_b']),
                 p['fh1_W'], r1(p['fh1_b']),
                 jnp.pad(p['fh2_W'], ((0, 0), (0, 127))), pad1(p['fh2_b']))
    lh, sh, fh = _k9_call(z, head_args)
    hom_pred = jnp.concatenate([lh[:, :1], sh[:, :1], fh[:, :1]], axis=1)

    return (adj, x_recon, y_logits, hom_pred, mu, logvar)


# speedup vs baseline: 5.0121x; 1.0994x over previous
import functools

import numpy as np
import jax
import jax.numpy as jnp
from jax import lax
from jax.experimental import pallas as pl
from jax.experimental.pallas import tpu as pltpu
from jax.experimental.pallas import tpu_sc as plsc

N = 10000
F = 256
LAT = 64
NCLS = 7

NC = 2
NS = 16
L = 16

_BN_S = float(1.0 / np.sqrt(1.0 + 1e-5))


NW = NC * NS
RPT = 320
AROWS = RPT + 8
NPAD = NW * RPT
ECHUNK = 128
SUP = 2688
MBUF = 288
def _cap(ep):
    return ep + ECHUNK

def _sc_mesh():
    return plsc.VectorSubcoreMesh(core_axis_name="c", subcore_axis_name="s")


def _scan_chunk(srcv, dstv, off, msrcv, mdstv, cnt, lo):
    lane = lax.iota(jnp.int32, L)
    for j in range(ECHUNK // L):
        d16 = dstv[pl.ds(off + j * L, L)]
        ld = d16 - lo
        ok = (ld >= 0) & (ld < RPT)
        inc = plsc.cumsum(ok.astype(jnp.int32))
        pos = jnp.where(ok, cnt + inc - 1, MBUF - L + lane)
        plsc.store_scatter(msrcv, [pos], srcv[pl.ds(off + j * L, L)])
        plsc.store_scatter(mdstv, [pos], ld)
        cnt = cnt + plsc.all_reduce_population_count(ok)
    return cnt


def _pad_tail(msrcv, mdstv, cnt):
    for g in range(ECHUNK // L):
        lane = lax.iota(jnp.int32, L) + g * L
        m = lane < cnt
        msrcv[pl.ds(g * L, L)] = jnp.where(m, msrcv[pl.ds(g * L, L)], 0)
        mdstv[pl.ds(g * L, L)] = jnp.where(m, mdstv[pl.ds(g * L, L)], RPT)


def _shift_tail(msrcv, mdstv):
    for g in range(L - ECHUNK // L):
        sl = pl.ds(ECHUNK + g * L, L)
        dl = pl.ds(g * L, L)
        msrcv[dl] = msrcv[sl]
        mdstv[dl] = mdstv[sl]


def _deg_flush(mdstv, acc):
    one0 = jnp.where(lax.iota(jnp.int32, L) == 0, 1.0, 0.0)
    cols = lax.iota(jnp.int32, L)

    @plsc.parallel_loop(0, ECHUNK, unroll=4)
    def _edges(e):
        row = plsc.load_gather(mdstv, [jnp.full((L,), e, jnp.int32)])
        plsc.addupdate_scatter(acc, [row, cols], one0)


def _deg_body(dst_hbm, src_hbm, zeros_hbm, out_hbm, srcl_hbm, dstl_hbm,
              cntl_hbm, srcv, dstv, msrcv, mdstv, cntbuf, acc, ssem, dsem):
    c = lax.axis_index("c")
    s = lax.axis_index("s")
    wid = c * NS + s
    lo = wid * RPT
    pltpu.sync_copy(zeros_hbm, acc)
    ep = dst_hbm.shape[0]

    def _emit(nf):
        _deg_flush(mdstv, acc)
        pltpu.sync_copy(msrcv.at[pl.ds(0, ECHUNK)],
                        srcl_hbm.at[wid, pl.ds(nf * ECHUNK, ECHUNK)])
        pltpu.sync_copy(mdstv.at[pl.ds(0, ECHUNK)],
                        dstl_hbm.at[wid, pl.ds(nf * ECHUNK, ECHUNK)])

    nsup = ep // SUP

    def _load_super(i, b):
        pltpu.async_copy(src_hbm.at[pl.ds(i * SUP, SUP)],
                         srcv.at[pl.ds(b * SUP, SUP)], ssem.at[b])
        pltpu.async_copy(dst_hbm.at[pl.ds(i * SUP, SUP)],
                         dstv.at[pl.ds(b * SUP, SUP)], dsem.at[b])

    _load_super(0, 0)

    @pl.loop(0, nsup, step=2, init_carry=(jnp.zeros((L,), jnp.int32), 0))
    def _supers(i0, carry):
        for b in range(2):
            i = i0 + b

            @pl.when(i + 1 < nsup)
            def _pf():
                _load_super(i + 1, 1 - b)

            pltpu.make_async_copy(src_hbm.at[pl.ds(i * SUP, SUP)],
                                  srcv.at[pl.ds(b * SUP, SUP)],
                                  ssem.at[b]).wait()
            pltpu.make_async_copy(dst_hbm.at[pl.ds(i * SUP, SUP)],
                                  dstv.at[pl.ds(b * SUP, SUP)],
                                  dsem.at[b]).wait()

            @pl.loop(0, SUP // ECHUNK, init_carry=carry)
            def _chunks(k, carry2):
                cnt, nf = carry2
                cnt = _scan_chunk(srcv, dstv, b * SUP + k * ECHUNK,
                                  msrcv, mdstv, cnt, lo)
                full = jnp.any(cnt >= ECHUNK)

                @pl.when(full)
                def _flush():
                    _emit(nf)
                    _shift_tail(msrcv, mdstv)

                return (jnp.where(full, cnt - ECHUNK, cnt),
                        jnp.where(full, nf + 1, nf))

            carry = _chunks
        return carry

    cnt, nf = _supers
    _pad_tail(msrcv, mdstv, cnt)
    _emit(nf)
    cntbuf[...] = jnp.full((L,), (nf + 1) * ECHUNK, jnp.int32)
    pltpu.sync_copy(cntbuf, cntl_hbm.at[wid])
    pltpu.sync_copy(acc.at[pl.ds(0, RPT)], out_hbm.at[pl.ds(lo, RPT)])


def _deg_call(src_pad, dst_pad, zeros_deg):
    ep = src_pad.shape[0]
    cap = _cap(ep)
    return pl.kernel(
        _deg_body,
        out_type=[jax.ShapeDtypeStruct((NPAD, L), jnp.float32),
                  jax.ShapeDtypeStruct((NW, cap), jnp.int32),
                  jax.ShapeDtypeStruct((NW, cap), jnp.int32),
                  jax.ShapeDtypeStruct((NW, L), jnp.int32)],
        mesh=_sc_mesh(),
        compiler_params=pltpu.CompilerParams(needs_layout_passes=False),
        scratch_types=[
            pltpu.VMEM((2 * SUP,), jnp.int32),
            pltpu.VMEM((2 * SUP,), jnp.int32),
            pltpu.VMEM((MBUF,), jnp.int32),
            pltpu.VMEM((MBUF,), jnp.int32),
            pltpu.VMEM((L,), jnp.int32),
            pltpu.VMEM((AROWS, L), jnp.float32),
            pltpu.SemaphoreType.DMA((2,)),
            pltpu.SemaphoreType.DMA((2,)),
        ],
    )(dst_pad, src_pad, zeros_deg)


BBLK = 64


def _acc_block(mdstv, rows, acc):
    cols = lax.iota(jnp.int32, L)

    @plsc.parallel_loop(0, BBLK, unroll=4)
    def _edges(e):
        row = plsc.load_gather(mdstv, [jnp.full((L,), e, jnp.int32)])
        for k in range(F // L):
            plsc.addupdate_scatter(acc, [row, cols + k * L],
                                   rows[e, pl.ds(k * L, L)])


def _agg_body(hw_hbm, srcl_hbm, dstl_hbm, cntl_hbm, zeros_hbm, out_hbm,
              cntv, mdstv, gidx, rows, acc, sem):
    c = lax.axis_index("c")
    s = lax.axis_index("s")
    wid = c * NS + s
    lo = wid * RPT
    pltpu.sync_copy(zeros_hbm, acc)
    pltpu.sync_copy(cntl_hbm.at[wid], cntv)
    cnt = cntv[...]
    cap = srcl_hbm.shape[1]
    nblk = cap // BBLK

    def _issue(i, b):
        pltpu.sync_copy(srcl_hbm.at[wid, pl.ds(i * BBLK, BBLK)], gidx.at[b])
        pltpu.sync_copy(dstl_hbm.at[wid, pl.ds(i * BBLK, BBLK)], mdstv.at[b])
        pltpu.async_copy(hw_hbm.at[gidx.at[b]], rows.at[b], sem.at[b])

    _issue(0, 0)

    @pl.loop(0, nblk, step=2)
    def _blocks(i0):
        for b in range(2):
            i = i0 + b
            nxt = (b + 1) % 2

            @pl.when(jnp.any((i + 1) * BBLK < cnt))
            def _prefetch():
                _issue(i + 1, nxt)

            @pl.when(jnp.any(i * BBLK < cnt))
            def _do():
                pltpu.make_async_copy(hw_hbm.at[gidx.at[b]], rows.at[b],
                                      sem.at[b]).wait()
                _acc_block(mdstv.at[b], rows.at[b], acc)

    pltpu.sync_copy(acc.at[pl.ds(0, RPT)], out_hbm.at[pl.ds(lo, RPT)])


def _agg_call(hw, srcl, dstl, cntl, zeros_acc):
    return pl.kernel(
        _agg_body,
        out_type=jax.ShapeDtypeStruct((NPAD, F), jnp.float32),
        mesh=_sc_mesh(),
        compiler_params=pltpu.CompilerParams(needs_layout_passes=False),
        scratch_types=[
            pltpu.VMEM((L,), jnp.int32),
            pltpu.VMEM((2, BBLK), jnp.int32),
            pltpu.VMEM((2, BBLK), jnp.int32),
            pltpu.VMEM((2, BBLK, F), jnp.float32),
            pltpu.VMEM((AROWS, F), jnp.float32),
            pltpu.SemaphoreType.DMA((2,)),
        ],
    )(hw, srcl, dstl, cntl, zeros_acc)


_RB = 1000


def _dis_of(degp_ref):
    deg = degp_ref[:, 0:1]
    return jnp.where(deg > 0.0, lax.rsqrt(deg), 0.0)


def _k2_body(x_ref, w_ref, degp_ref, out_ref):
    dis = _dis_of(degp_ref)
    out_ref[...] = jnp.dot(x_ref[...], w_ref[...],
                           preferred_element_type=jnp.float32) * dis


def _k2_call(x, w, degp):
    grid = (N // _RB,)
    return pl.pallas_call(
        _k2_body,
        grid=grid,
        in_specs=[
            pl.BlockSpec((_RB, F), lambda r: (r, 0)),
            pl.BlockSpec((F, F), lambda r: (0, 0)),
            pl.BlockSpec((_RB, L), lambda r: (r, 0)),
        ],
        out_specs=pl.BlockSpec((_RB, F), lambda r: (r, 0)),
        out_shape=jax.ShapeDtypeStruct((N, F), jnp.float32),
    )(x, w, degp)


def _post_h(hom_ref, agg_ref, degp_ref, b_ref, g_ref, bb_ref, hw_ref, hb_ref):
    dis = _dis_of(degp_ref)
    gcn = agg_ref[...] * dis + b_ref[...]
    h = jnp.maximum(gcn * _BN_S * g_ref[...] + bb_ref[...], 0.0)
    hv = (hom_ref[0] * hw_ref[0:1, :] + hom_ref[1] * hw_ref[1:2, :]
          + hom_ref[2] * hw_ref[2:3, :] + hb_ref[...])
    return h + hv, dis


def _k4_body(hom_ref, agg_ref, degp_ref, b_ref, g_ref, bb_ref, hw_ref, hb_ref,
             w2_ref, out_ref):
    h, dis = _post_h(hom_ref, agg_ref, degp_ref, b_ref, g_ref, bb_ref,
                     hw_ref, hb_ref)
    out_ref[...] = jnp.dot(h, w2_ref[...],
                           preferred_element_type=jnp.float32) * dis


def _k4_call(hom3, agg, degp, b, g, bb, hw, hb, w2):
    grid = (N // _RB,)
    row = lambda r: (r, 0)
    full2 = lambda r: (0, 0)
    return pl.pallas_call(
        _k4_body,
        grid=grid,
        in_specs=[
            pl.BlockSpec(memory_space=pltpu.SMEM),
            pl.BlockSpec((_RB, F), row),
            pl.BlockSpec((_RB, L), lambda r: (r, 0)),
            pl.BlockSpec((1, F), full2),
            pl.BlockSpec((1, F), full2),
            pl.BlockSpec((1, F), full2),
            pl.BlockSpec((3, F), full2),
            pl.BlockSpec((1, F), full2),
            pl.BlockSpec((F, F), full2),
        ],
        out_specs=pl.BlockSpec((_RB, F), row),
        out_shape=jax.ShapeDtypeStruct((N, F), jnp.float32),
    )(hom3, agg, degp, b, g, bb, hw, hb, w2)


def _k6_body(hom_ref, agg_ref, degp_ref, b_ref, g_ref, bb_ref, hw_ref, hb_ref,
             muw_ref, muw1_ref, mub_ref, lvw_ref, lvw1_ref, lvb_ref,
             mu_ref, lv_ref):
    h, _ = _post_h(hom_ref, agg_ref, degp_ref, b_ref, g_ref, bb_ref,
                   hw_ref, hb_ref)
    mucst = (hom_ref[0] * muw1_ref[0:1, :] + hom_ref[1] * muw1_ref[1:2, :]
             + hom_ref[2] * muw1_ref[2:3, :] + mub_ref[...])
    lvcst = (hom_ref[0] * lvw1_ref[0:1, :] + hom_ref[1] * lvw1_ref[1:2, :]
             + hom_ref[2] * lvw1_ref[2:3, :] + lvb_ref[...])
    mu_ref[...] = jnp.dot(h, muw_ref[...],
                          preferred_element_type=jnp.float32) + mucst
    lv_ref[...] = jnp.dot(h, lvw_ref[...],
                          preferred_element_type=jnp.float32) + lvcst


def _k6_call(hom3, agg, degp, b, g, bb, hw, hb, muw, muw1, mub, lvw, lvw1, lvb):
    grid = (N // _RB,)
    row = lambda r: (r, 0)
    full2 = lambda r: (0, 0)
    return pl.pallas_call(
        _k6_body,
        grid=grid,
        in_specs=[
            pl.BlockSpec(memory_space=pltpu.SMEM),
            pl.BlockSpec((_RB, F), row),
            pl.BlockSpec((_RB, L), lambda r: (r, 0)),
            pl.BlockSpec((1, F), full2),
            pl.BlockSpec((1, F), full2),
            pl.BlockSpec((1, F), full2),
            pl.BlockSpec((3, F), full2),
            pl.BlockSpec((1, F), full2),
            pl.BlockSpec((F, LAT), full2),
            pl.BlockSpec((3, LAT), full2),
            pl.BlockSpec((1, LAT), full2),
            pl.BlockSpec((F, LAT), full2),
            pl.BlockSpec((3, LAT), full2),
            pl.BlockSpec((1, LAT), full2),
        ],
        out_specs=[pl.BlockSpec((_RB, LAT), row), pl.BlockSpec((_RB, LAT), row)],
        out_shape=[jax.ShapeDtypeStruct((N, LAT), jnp.float32),
                   jax.ShapeDtypeStruct((N, LAT), jnp.float32)],
    )(hom3, agg, degp, b, g, bb, hw, hb, muw, muw1, mub, lvw, lvw1, lvb)


_ARB = 512
_ACB = 2048


def _adj_body(zi_ref, zj_ref, out_ref):
    prod = lax.dot_general(zi_ref[...], zj_ref[...], (((1,), (1,)), ((), ())),
                           preferred_element_type=jnp.float32)
    out_ref[...] = jax.nn.sigmoid(prod)


def _adj_call(z):
    grid = (pl.cdiv(N, _ARB), pl.cdiv(N, _ACB))
    return pl.pallas_call(
        _adj_body,
        grid=grid,
        in_specs=[
            pl.BlockSpec((_ARB, LAT), lambda r, c: (r, 0)),
            pl.BlockSpec((_ACB, LAT), lambda r, c: (c, 0)),
        ],
        out_specs=pl.BlockSpec((_ARB, _ACB), lambda r, c: (r, c)),
        out_shape=jax.ShapeDtypeStruct((N, N), jnp.float32),
    )(z, z)


def _k8_body(z_ref, pw_ref, pb_ref, pg_ref, pbb_ref, t1w_ref, t1b_ref,
             t2w_ref, t2b_ref, l1w_ref, l1b_ref, l2w_ref, l2b_ref,
             xr_ref, yl_ref):
    z = z_ref[...]
    zp = (jnp.dot(z, pw_ref[...], preferred_element_type=jnp.float32)
          + pb_ref[...]) * _BN_S * pg_ref[...] + pbb_ref[...]
    t = jnp.maximum(jnp.dot(zp, t1w_ref[...],
                            preferred_element_type=jnp.float32) + t1b_ref[...], 0.0)
    xr_ref[...] = jnp.dot(t, t2w_ref[...],
                          preferred_element_type=jnp.float32) + t2b_ref[...]
    u = jnp.maximum(jnp.dot(z, l1w_ref[...],
                            preferred_element_type=jnp.float32) + l1b_ref[...], 0.0)
    yl_ref[...] = jnp.dot(u, l2w_ref[...],
                          preferred_element_type=jnp.float32) + l2b_ref[...]


def _k8_call(z, pw, pb, pg, pbb, t1w, t1b, t2w, t2b, l1w, l1b, l2w, l2b):
    grid = (N // _RB,)
    row = lambda r: (r, 0)
    full2 = lambda r: (0, 0)
    tl = pw.shape[1]
    return pl.pallas_call(
        _k8_body,
        grid=grid,
        in_specs=[
            pl.BlockSpec((_RB, LAT), row),
            pl.BlockSpec((LAT, tl), full2),
            pl.BlockSpec((1, tl), full2),
            pl.BlockSpec((1, tl), full2),
            pl.BlockSpec((1, tl), full2),
            pl.BlockSpec((tl, F), full2),
            pl.BlockSpec((1, F), full2),
            pl.BlockSpec((F, F), full2),
            pl.BlockSpec((1, F), full2),
            pl.BlockSpec((LAT, LAT), full2),
            pl.BlockSpec((1, LAT), full2),
            pl.BlockSpec((LAT, 128), full2),
            pl.BlockSpec((1, 128), full2),
        ],
        out_specs=[pl.BlockSpec((_RB, F), row), pl.BlockSpec((_RB, 128), row)],
        out_shape=[jax.ShapeDtypeStruct((N, F), jnp.float32),
                   jax.ShapeDtypeStruct((N, 128), jnp.float32)],
    )(z, pw, pb, pg, pbb, t1w, t1b, t2w, t2b, l1w, l1b, l2w, l2b)


def _k9_body(z_ref, lh1_ref, lh1b_ref, lh2_ref, lh2b_ref,
             sh1_ref, sh1b_ref, sh2_ref, sh2b_ref,
             fh1_ref, fh1b_ref, fh2_ref, fh2b_ref,
             lh_ref, sh_ref, fh_ref):
    zg = jnp.sum(z_ref[...], axis=0, keepdims=True) * (1.0 / N)

    def head(w1, b1, w2, b2):
        u = jnp.maximum(jnp.dot(zg, w1[...],
                                preferred_element_type=jnp.float32) + b1[...], 0.0)
        return jnp.dot(u, w2[...], preferred_element_type=jnp.float32) + b2[...]

    lh_ref[...] = jax.nn.sigmoid(head(lh1_ref, lh1b_ref, lh2_ref, lh2b_ref))
    sh_ref[...] = jax.nn.sigmoid(head(sh1_ref, sh1b_ref, sh2_ref, sh2b_ref))
    fh_ref[...] = jnp.tanh(head(fh1_ref, fh1b_ref, fh2_ref, fh2b_ref))


def _k9_call(z, args):
    specs = [pl.BlockSpec((N, LAT), lambda: (0, 0))]
    for a in args:
        specs.append(pl.BlockSpec(a.shape, lambda: (0, 0)))
    return pl.pallas_call(
        _k9_body,
        in_specs=specs,
        out_specs=[pl.BlockSpec((1, 128), lambda: (0, 0))] * 3,
        out_shape=[jax.ShapeDtypeStruct((1, 128), jnp.float32)] * 3,
    )(z, *args)


def kernel(x, edge_index, homophily_cond, batch, params):
    p = params
    e = edge_index.shape[1]
    loop = jnp.arange(N, dtype=jnp.int32)
    src = jnp.concatenate([edge_index[0].astype(jnp.int32), loop])
    dst = jnp.concatenate([edge_index[1].astype(jnp.int32), loop])
    ep = ((e + N + SUP - 1) // SUP) * SUP
    src = jnp.pad(src, (0, ep - e - N))
    dst = jnp.pad(dst, (0, ep - e - N), constant_values=N)

    zeros_deg = jnp.zeros((AROWS, L), jnp.float32)
    zeros_acc = jnp.zeros((AROWS, F), jnp.float32)
    hom3 = homophily_cond[0]

    r1 = lambda a: a.reshape(1, -1)

    degp, srcl, dstl, cntl = _deg_call(src, dst, zeros_deg)

    hw1 = _k2_call(x, p['gcn0_W'], degp)
    agg1 = _agg_call(hw1, srcl, dstl, cntl, zeros_acc)[:N]

    hw2 = _k4_call(hom3, agg1, degp, r1(p['gcn0_b']), r1(p['bn0_g']),
                   r1(p['bn0_b']), p['hom0_W'], r1(p['hom0_b']), p['gcn1_W'])
    agg2 = _agg_call(hw2, srcl, dstl, cntl, zeros_acc)[:N]

    muw0, muw1 = p['mu_W'][:F], p['mu_W'][F:]
    lvw0, lvw1 = p['lv_W'][:F], p['lv_W'][F:]
    mu, logvar = _k6_call(hom3, agg2, degp, r1(p['gcn1_b']), r1(p['bn1_g']),
                          r1(p['bn1_b']), p['hom1_W'], r1(p['hom1_b']),
                          muw0, muw1, r1(p['mu_b']), lvw0, lvw1, r1(p['lv_b']))
    z = mu

    adj = _adj_call(z)

    lab2w = jnp.pad(p['lab2_W'], ((0, 0), (0, 128 - NCLS)))
    lab2b = jnp.pad(r1(p['lab2_b']), ((0, 0), (0, 128 - NCLS)))
    x_recon, ylp = _k8_call(z, p['proj_W'], r1(p['proj_b']), r1(p['projbn_g']),
                            r1(p['projbn_b']), p['t1_W'], r1(p['t1_b']),
                            p['t2_W'], r1(p['t2_b']), p['lab1_W'],
                            r1(p['lab1_b']), lab2w, lab2b)
    y_logits = ylp[:, :NCLS]

    pad1 = lambda a: jnp.pad(r1(a), ((0, 0), (0, 128 - a.reshape(-1).shape[0])))
    head_args = (p['lh1_W'], r1(p['lh1_b']),
                 jnp.pad(p['lh2_W'], ((0, 0), (0, 127))), pad1(p['lh2_b']),
                 p['sh1_W'], r1(p['sh1_b']),
                 jnp.pad(p['sh2_W'], ((0, 0), (0, 127))), pad1(p['sh2_b']),
                 p['fh1_W'], r1(p['fh1_b']),
                 jnp.pad(p['fh2_W'], ((0, 0), (0, 127))), pad1(p['fh2_b']))
    lh, sh, fh = _k9_call(z, head_args)
    hom_pred = jnp.concatenate([lh[:, :1], sh[:, :1], fh[:, :1]], axis=1)

    return (adj, x_recon, y_logits, hom_pred, mu, logvar)
